# Initial kernel scaffold; baseline (speedup 1.0000x reference)
import jax
import jax.numpy as jnp
from jax.experimental import pallas as pl


def kernel(x, edge_index, edge_weight, batch, cover_node, cover_cluster, edge_index2, edge_weight2, batch2, cW1, cb1, cW2, cb2, cWl, cbl, bW1, bb1, bW2, bb2, bWl, bbl, gamma, beta, l1W, l1b, l2W, l2b):
    raise NotImplementedError("write your pallas kernel here")


# speedup vs baseline: 4.2118x; 4.2118x over previous
import functools

import jax
import jax.numpy as jnp
from jax import lax
from jax.experimental import pallas as pl
from jax.experimental.pallas import tpu as pltpu
from jax.experimental.pallas import tpu_sc as plsc

_N0, _E0, _D, _H, _B = 10000, 320000, 128, 64, 16
_C, _N1, _E1, _NC = 15000, 2500, 80000, 10
_NCORE, _NSUB, _NW = 2, 16, 32
_K = 512
_CK = 256
_CLPT = 80
_R = 512

_EP0 = 327680
_EP1 = 81920
_CP = 15360

_f32 = jnp.float32
_i32 = jnp.int32


def _iota16():
    return lax.iota(_i32, 16)


def _zero2d(ref, nrows, ncols):
    z = jnp.zeros((16,), _f32)
    it = _iota16()

    def zrow(j, carry):
        rows16 = it + j * 16
        for c in range(ncols):
            plsc.store_scatter(ref, [rows16, jnp.full((16,), c, _i32)], z)
        return carry
    lax.fori_loop(0, nrows // 16, zrow, 0)


_SC_PARAMS = pltpu.CompilerParams(needs_layout_passes=False,
                                  use_tc_tiling_on_sc=False)


def _round_up(a, m):
    return ((a + m - 1) // m) * m


def _make_deg_kernel(n, e_pad):
    npad = _round_up(n, _NSUB * 8)
    rps = npad // _NSUB
    ept = e_pad // _NW
    nchunks = ept // _K
    mesh = plsc.VectorSubcoreMesh(core_axis_name="c", subcore_axis_name="s")

    @functools.partial(
        pl.kernel,
        out_type=jax.ShapeDtypeStruct((_NCORE, npad, 16), _f32),
        mesh=mesh,
        compiler_params=_SC_PARAMS,
        scratch_types=[
            pltpu.VMEM((_K,), _i32),
            pltpu.VMEM((_K,), _f32),
            pltpu.VMEM((_K, 16), _f32),
            pltpu.VMEM_SHARED((npad, 16), _f32),
        ],
    )
    def deg_kernel(d_hbm, w_hbm, out_hbm, didx_v, w_v, rows_v, acc_sh):
        cid = lax.axis_index("c")
        sid = lax.axis_index("s")
        wid = sid * _NCORE + cid

        _zero2d(rows_v, _K, 16)
        off = 0
        while off < rps:
            csz = min(_K, rps - off)
            pltpu.sync_copy(rows_v.at[pl.ds(0, csz)],
                            acc_sh.at[pl.ds(sid * rps + off, csz)])
            off += csz
        plsc.subcore_barrier()
        it = _iota16()

        def chunk(k, carry):
            base = wid * ept + k * _K
            pltpu.sync_copy(d_hbm.at[pl.ds(base, _K)], didx_v)
            pltpu.sync_copy(w_hbm.at[pl.ds(base, _K)], w_v)

            def grp(j, carry2):
                w16 = w_v[pl.ds(j * 16, 16)]
                rows16 = it + j * 16
                for c in range(16):
                    plsc.store_scatter(
                        rows_v, [rows16, jnp.full((16,), c, _i32)], w16)
                return carry2
            lax.fori_loop(0, _K // 16, grp, 0)
            pltpu.sync_copy(rows_v, acc_sh.at[didx_v], add=True)
            return carry
        lax.fori_loop(0, nchunks, chunk, 0)
        plsc.subcore_barrier()

        off = 0
        while off < rps:
            csz = min(_K, rps - off)
            pltpu.sync_copy(acc_sh.at[pl.ds(sid * rps + off, csz)],
                            out_hbm.at[cid, pl.ds(sid * rps + off, csz)])
            off += csz

    return deg_kernel


def _make_msg_kernel(n, e_pad):
    npad = _round_up(n, _NSUB * 8)
    rps = npad // _NSUB
    ept = e_pad // _NW
    nchunks = ept // _K
    mesh = plsc.VectorSubcoreMesh(core_axis_name="c", subcore_axis_name="s")

    @functools.partial(
        pl.kernel,
        out_type=jax.ShapeDtypeStruct((_NCORE, npad, _H), _f32),
        mesh=mesh,
        compiler_params=_SC_PARAMS,
        scratch_types=[
            pltpu.VMEM((_K,), _i32),
            pltpu.VMEM((_K,), _i32),
            pltpu.VMEM((_K,), _f32),
            pltpu.VMEM((_K, _H), _f32),
            pltpu.VMEM_SHARED((npad, _H), _f32),
            pltpu.SemaphoreType.DMA,
        ],
    )
    def msg_kernel(g_hbm, s_hbm, d_hbm, w_hbm, out_hbm,
                   sidx_v, didx_v, w_v, rows_v, acc_sh, sem):
        cid = lax.axis_index("c")
        sid = lax.axis_index("s")
        wid = sid * _NCORE + cid

        _zero2d(rows_v, _K, _H)
        off = 0
        while off < rps:
            csz = min(_K, rps - off)
            pltpu.sync_copy(rows_v.at[pl.ds(0, csz)],
                            acc_sh.at[pl.ds(sid * rps + off, csz)])
            off += csz
        plsc.subcore_barrier()
        it = _iota16()

        def chunk(k, carry):
            base = wid * ept + k * _K
            pltpu.sync_copy(s_hbm.at[pl.ds(base, _K)], sidx_v)
            pltpu.sync_copy(d_hbm.at[pl.ds(base, _K)], didx_v)
            pltpu.sync_copy(w_hbm.at[pl.ds(base, _K)], w_v)
            pltpu.async_copy(g_hbm.at[sidx_v], rows_v, sem).wait()

            def grp(j, carry2):
                w16 = w_v[pl.ds(j * 16, 16)]
                rows16 = it + j * 16
                for c in range(_H):
                    colv = jnp.full((16,), c, _i32)
                    vals = plsc.load_gather(rows_v, [rows16, colv])
                    plsc.store_scatter(rows_v, [rows16, colv], vals * w16)
                return carry2
            lax.fori_loop(0, _K // 16, grp, 0)
            pltpu.sync_copy(rows_v, acc_sh.at[didx_v], add=True)
            return carry
        lax.fori_loop(0, nchunks, chunk, 0)
        plsc.subcore_barrier()

        off = 0
        while off < rps:
            csz = min(_K, rps - off)
            pltpu.sync_copy(acc_sh.at[pl.ds(sid * rps + off, csz)],
                            out_hbm.at[cid, pl.ds(sid * rps + off, csz)])
            off += csz

    return msg_kernel


def _make_cover_kernel():
    nout = _NW * _CLPT
    mesh = plsc.VectorSubcoreMesh(core_axis_name="c", subcore_axis_name="s")

    @functools.partial(
        pl.kernel,
        out_type=(jax.ShapeDtypeStruct((nout, _H), _f32),
                  jax.ShapeDtypeStruct((nout, _H), _f32)),
        mesh=mesh,
        compiler_params=_SC_PARAMS,
        scratch_types=[
            pltpu.VMEM((_CK,), _i32),
            pltpu.VMEM((_CK,), _i32),
            pltpu.VMEM((_CK, _H), _f32),
            pltpu.VMEM((_CLPT, _H), _f32),
            pltpu.VMEM((_CLPT, _H), _f32),
            pltpu.SemaphoreType.DMA,
        ],
    )
    def cover_kernel(h_hbm, cc_hbm, cn_hbm, out_add, out_max,
                     cc_v, cn_v, rows_v, asum, amax, sem):
        cid = lax.axis_index("c")
        sid = lax.axis_index("s")
        wid = sid * _NCORE + cid
        c_lo = wid * _CLPT
        c_hi = jnp.minimum(c_lo + _CLPT, _N1)

        def cnt_chunk(k, carry):
            lo, hi = carry
            pltpu.sync_copy(cc_hbm.at[pl.ds(k * _CK, _CK)], cc_v)

            def cnt_grp(j, carry2):
                lo2, hi2 = carry2
                v = cc_v[pl.ds(j * 16, 16)]
                lo2 = lo2 + plsc.all_reduce_population_count(v < c_lo)
                hi2 = hi2 + plsc.all_reduce_population_count(v < c_hi)
                return lo2, hi2
            return lax.fori_loop(0, _CK // 16, cnt_grp, (lo, hi))
        z16 = jnp.zeros((16,), _i32)
        lo_v, hi_v = lax.fori_loop(0, _CP // _CK, cnt_chunk, (z16, z16))
        e_lo = jnp.max(lo_v)
        e_hi = jnp.max(hi_v)
        estart = (e_lo // 8) * 8
        nch = (e_hi - estart + _CK - 1) // _CK

        _zero2d(asum, _CLPT, _H)
        _zero2d(amax, _CLPT, _H)
        it = _iota16()

        def chunk(k, carry):
            base = estart + k * _CK
            pltpu.sync_copy(cc_hbm.at[pl.ds(base, _CK)], cc_v)
            pltpu.sync_copy(cn_hbm.at[pl.ds(base, _CK)], cn_v)
            pltpu.async_copy(h_hbm.at[cn_v], rows_v, sem).wait()

            def ent(j, carry2):
                ci16 = cc_v[pl.ds(j * 16, 16)]
                for i in range(16):
                    ci = ci16[i]
                    inr = jnp.logical_and(ci >= c_lo, ci < c_hi)
                    l = jnp.clip(ci - c_lo, 0, _CLPT - 1)
                    m = jnp.where(inr, jnp.float32(1.0), jnp.float32(0.0))
                    rsel = jnp.full((16,), j * 16 + i, _i32)
                    lsel = jnp.full((16,), l, _i32)
                    for c in range(_H // 16):
                        cols = it + c * 16
                        val = plsc.load_gather(rows_v, [rsel, cols]) * m
                        so = plsc.load_gather(asum, [lsel, cols])
                        plsc.store_scatter(asum, [lsel, cols], so + val)
                        mo = plsc.load_gather(amax, [lsel, cols])
                        plsc.store_scatter(amax, [lsel, cols],
                                           jnp.maximum(mo, val))
                return carry2
            lax.fori_loop(0, _CK // 16, ent, 0)
            return carry
        lax.fori_loop(0, nch, chunk, 0)

        pltpu.sync_copy(asum, out_add.at[pl.ds(c_lo, _CLPT)])
        pltpu.sync_copy(amax, out_max.at[pl.ds(c_lo, _CLPT)])

    return cover_kernel


def _mm_first_body(x_ref, w_ref, degp_ref, g_ref, dinv_ref):
    dinv = lax.rsqrt(1.0 + degp_ref[0, :, 0:1] + degp_ref[1, :, 0:1])
    g_ref[...] = dinv * jnp.dot(x_ref[...], w_ref[...],
                                preferred_element_type=_f32)
    dinv_ref[...] = dinv


def _mm_first(x, w, degp, n):
    grid = pl.cdiv(n, _R)
    return pl.pallas_call(
        _mm_first_body,
        grid=(grid,),
        in_specs=[
            pl.BlockSpec((_R, x.shape[1]), lambda i: (i, 0)),
            pl.BlockSpec(w.shape, lambda i: (0, 0)),
            pl.BlockSpec((_NCORE, _R, 16), lambda i: (0, i, 0)),
        ],
        out_specs=[
            pl.BlockSpec((_R, _H), lambda i: (i, 0)),
            pl.BlockSpec((_R, 1), lambda i: (i, 0)),
        ],
        out_shape=[
            jax.ShapeDtypeStruct((n, _H), _f32),
            jax.ShapeDtypeStruct((n, 1), _f32),
        ],
    )(x, w, degp)


def _mm_pair_body(xa_ref, xb_ref, wt_ref, wb_ref, degp_ref, g_ref, dinv_ref):
    dinv = lax.rsqrt(1.0 + degp_ref[0, :, 0:1] + degp_ref[1, :, 0:1])
    acc = (jnp.dot(xa_ref[...], wt_ref[...], preferred_element_type=_f32)
           + jnp.dot(xb_ref[...], wb_ref[...], preferred_element_type=_f32))
    g_ref[...] = dinv * acc
    dinv_ref[...] = dinv


def _mm_pair(xa, xb, wt, wb, degp, n):
    grid = pl.cdiv(n, _R)
    return pl.pallas_call(
        _mm_pair_body,
        grid=(grid,),
        in_specs=[
            pl.BlockSpec((_R, _H), lambda i: (i, 0)),
            pl.BlockSpec((_R, _H), lambda i: (i, 0)),
            pl.BlockSpec((_H, _H), lambda i: (0, 0)),
            pl.BlockSpec((_H, _H), lambda i: (0, 0)),
            pl.BlockSpec((_NCORE, _R, 16), lambda i: (0, i, 0)),
        ],
        out_specs=[
            pl.BlockSpec((_R, _H), lambda i: (i, 0)),
            pl.BlockSpec((_R, 1), lambda i: (i, 0)),
        ],
        out_shape=[
            jax.ShapeDtypeStruct((n, _H), _f32),
            jax.ShapeDtypeStruct((n, 1), _f32),
        ],
    )(xa, xb, wt, wb, degp)


def _mid_body(msgp_ref, g_ref, dinv_ref, b_ref, w_ref, h_ref, g2_ref):
    dinv = dinv_ref[...]
    h = jnp.maximum(
        dinv * (msgp_ref[0] + msgp_ref[1] + g_ref[...]) + b_ref[...], 0.0)
    h_ref[...] = h
    g2_ref[...] = dinv * jnp.dot(h, w_ref[...], preferred_element_type=_f32)


def _mid(msgp, g, dinv, b, w, n):
    grid = pl.cdiv(n, _R)
    return pl.pallas_call(
        _mid_body,
        grid=(grid,),
        in_specs=[
            pl.BlockSpec((_NCORE, _R, _H), lambda i: (0, i, 0)),
            pl.BlockSpec((_R, _H), lambda i: (i, 0)),
            pl.BlockSpec((_R, 1), lambda i: (i, 0)),
            pl.BlockSpec((1, _H), lambda i: (0, 0)),
            pl.BlockSpec((_H, _H), lambda i: (0, 0)),
        ],
        out_specs=[
            pl.BlockSpec((_R, _H), lambda i: (i, 0)),
            pl.BlockSpec((_R, _H), lambda i: (i, 0)),
        ],
        out_shape=[
            jax.ShapeDtypeStruct((n, _H), _f32),
            jax.ShapeDtypeStruct((n, _H), _f32),
        ],
    )(msgp, g, dinv, b, w)


def _blockout_body(msgp_ref, g2_ref, dinv_ref, b2_ref, h1_ref,
                   wlt_ref, wlb_ref, bl_ref, h_ref):
    dinv = dinv_ref[...]
    h2 = jnp.maximum(
        dinv * (msgp_ref[0] + msgp_ref[1] + g2_ref[...]) + b2_ref[...], 0.0)
    acc = (jnp.dot(h1_ref[...], wlt_ref[...], preferred_element_type=_f32)
           + jnp.dot(h2, wlb_ref[...], preferred_element_type=_f32))
    h_ref[...] = jnp.maximum(acc + bl_ref[...], 0.0)


def _blockout(msgp, g2, dinv, b2, h1, wlt, wlb, bl, n):
    grid = pl.cdiv(n, _R)
    return pl.pallas_call(
        _blockout_body,
        grid=(grid,),
        in_specs=[
            pl.BlockSpec((_NCORE, _R, _H), lambda i: (0, i, 0)),
            pl.BlockSpec((_R, _H), lambda i: (i, 0)),
            pl.BlockSpec((_R, 1), lambda i: (i, 0)),
            pl.BlockSpec((1, _H), lambda i: (0, 0)),
            pl.BlockSpec((_R, _H), lambda i: (i, 0)),
            pl.BlockSpec((_H, _H), lambda i: (0, 0)),
            pl.BlockSpec((_H, _H), lambda i: (0, 0)),
            pl.BlockSpec((1, _H), lambda i: (0, 0)),
        ],
        out_specs=pl.BlockSpec((_R, _H), lambda i: (i, 0)),
        out_shape=jax.ShapeDtypeStruct((n, _H), _f32),
    )(msgp, g2, dinv, b2, h1, wlt, wlb, bl)


def _pool_body(n, h_ref, bidx_ref, sum_ref, max_ref):
    i = pl.program_id(0)
    h = h_ref[...]
    bid = bidx_ref[...]
    row = i * _R + lax.broadcasted_iota(_i32, (_R, 1), 0)
    valid = row < n
    onehot = jnp.logical_and(
        bid == lax.broadcasted_iota(_i32, (1, _B), 1), valid).astype(_f32)
    s = lax.dot_general(onehot, h, (((0,), (0,)), ((), ())),
                        preferred_element_type=_f32)
    rows = []
    for b in range(_B):
        m = jnp.where(jnp.logical_and(bid == b, valid), h, 0.0)
        rows.append(jnp.max(m, axis=0, keepdims=True))
    mx = jnp.concatenate(rows, axis=0)

    @pl.when(i == 0)
    def _():
        sum_ref[...] = s
        max_ref[...] = mx

    @pl.when(i > 0)
    def _():
        sum_ref[...] = sum_ref[...] + s
        max_ref[...] = jnp.maximum(max_ref[...], mx)


def _pool(h, bidx, n):
    grid = pl.cdiv(n, _R)
    return pl.pallas_call(
        functools.partial(_pool_body, n),
        grid=(grid,),
        in_specs=[
            pl.BlockSpec((_R, _H), lambda i: (i, 0)),
            pl.BlockSpec((_R, 1), lambda i: (i, 0)),
        ],
        out_specs=[
            pl.BlockSpec((_B, _H), lambda i: (0, 0)),
            pl.BlockSpec((_B, _H), lambda i: (0, 0)),
        ],
        out_shape=[
            jax.ShapeDtypeStruct((_B, _H), _f32),
            jax.ShapeDtypeStruct((_B, _H), _f32),
        ],
    )(h, bidx)


def _head_body(x0, x1, x2, x3, gamma, beta, w1, b1, w2, b2, out):
    z = jnp.concatenate([x0[...], x1[...], x2[...], x3[...]], axis=1)
    mu = jnp.mean(z, axis=0, keepdims=True)
    var = jnp.mean((z - mu) ** 2, axis=0, keepdims=True)
    z = (z - mu) * lax.rsqrt(var + 1e-5) * gamma[...] + beta[...]
    z = jnp.maximum(jnp.dot(z, w1[...], preferred_element_type=_f32)
                    + b1[...], 0.0)
    z = jnp.dot(z, w2[...], preferred_element_type=_f32) + b2[...]
    z = z - jnp.max(z, axis=1, keepdims=True)
    ez = jnp.exp(z)
    out[...] = ez / jnp.sum(ez, axis=1, keepdims=True)


def _head(x0, x1, x2, x3, gamma, beta, w1, b1, w2, b2):
    return pl.pallas_call(
        _head_body,
        out_shape=jax.ShapeDtypeStruct((_B, _NC), _f32),
    )(x0, x1, x2, x3, gamma, beta, w1, b1, w2, b2)


_deg0 = _make_deg_kernel(_N0, _EP0)
_deg1 = _make_deg_kernel(_N1, _EP1)
_msg0 = _make_msg_kernel(_N0, _EP0)
_msg1 = _make_msg_kernel(_N1, _EP1)
_cover = _make_cover_kernel()


def kernel(x, edge_index, edge_weight, batch, cover_node, cover_cluster,
           edge_index2, edge_weight2, batch2, cW1, cb1, cW2, cb2, cWl, cbl,
           bW1, bb1, bW2, bb2, bWl, bbl, gamma, beta, l1W, l1b, l2W, l2b):
    s0 = jnp.pad(edge_index[0], (0, _EP0 - _E0))
    d0 = jnp.pad(edge_index[1], (0, _EP0 - _E0))
    w0 = jnp.pad(edge_weight, (0, _EP0 - _E0))
    s1 = jnp.pad(edge_index2[0], (0, _EP1 - _E1))
    d1 = jnp.pad(edge_index2[1], (0, _EP1 - _E1))
    w1 = jnp.pad(edge_weight2, (0, _EP1 - _E1))
    ccp = jnp.pad(cover_cluster, (0, _CP - _C), constant_values=1 << 30)
    cnp = jnp.pad(cover_node, (0, _CP - _C))

    degp0 = _deg0(d0, w0)
    degp1 = _deg1(d1, w1)

    g1, dinv0 = _mm_first(x, cW1, degp0, _N0)
    mp = _msg0(g1, s0, d0, w0)
    h1, g2 = _mid(mp, g1, dinv0, cb1.reshape(1, _H), cW2, _N0)
    mp = _msg0(g2, s0, d0, w0)
    h = _blockout(mp, g2, dinv0, cb2.reshape(1, _H), h1,
                  cWl[:_H], cWl[_H:], cbl.reshape(1, _H), _N0)

    xs0, xs1 = _pool(h, batch.reshape(_N0, 1), _N0)
    xadd_p, xmax_p = _cover(h, ccp, cnp)
    x_add = xadd_p[:_N1]
    x_max = xmax_p[:_N1]

    gB, dinv1 = _mm_pair(x_add, x_max, bW1[:_H], bW1[_H:], degp1, _N1)
    mp = _msg1(gB, s1, d1, w1)
    h1B, g2B = _mid(mp, gB, dinv1, bb1.reshape(1, _H), bW2, _N1)
    mp = _msg1(g2B, s1, d1, w1)
    hB = _blockout(mp, g2B, dinv1, bb2.reshape(1, _H), h1B,
                   bWl[:_H], bWl[_H:], bbl.reshape(1, _H), _N1)

    xs2, xs3 = _pool(hB, batch2.reshape(_N1, 1), _N1)

    return _head(xs0, xs1, xs2, xs3,
                 gamma.reshape(1, 4 * _H), beta.reshape(1, 4 * _H),
                 l1W, l1b.reshape(1, _H), l2W, l2b.reshape(1, _NC))


# speedup vs baseline: 4.8825x; 1.1592x over previous
import functools

import jax
import jax.numpy as jnp
from jax import lax
from jax.experimental import pallas as pl
from jax.experimental.pallas import tpu as pltpu
from jax.experimental.pallas import tpu_sc as plsc

_N0, _E0, _D, _H, _B = 10000, 320000, 128, 64, 16
_C, _N1, _E1, _NC = 15000, 2500, 80000, 10
_NCORE, _NSUB, _NW = 2, 16, 32
_K = 512
_CK = 256
_CLPT = 80
_R = 512

_EP0 = 327680
_EP1 = 81920
_CP = 15360
_NP0 = 10112
_NP1 = 2560

_f32 = jnp.float32
_i32 = jnp.int32


def _iota16():
    return lax.iota(_i32, 16)


def _zero2d(ref, nrows, ncols):
    z = jnp.zeros((16,), _f32)
    it = _iota16()

    def zrow(j, carry):
        rows16 = it + j * 16
        for c in range(ncols):
            plsc.store_scatter(ref, [rows16, jnp.full((16,), c, _i32)], z)
        return carry
    lax.fori_loop(0, nrows // 16, zrow, 0)


_SC_PARAMS = pltpu.CompilerParams(needs_layout_passes=False,
                                  use_tc_tiling_on_sc=False)


def _round_up(a, m):
    return ((a + m - 1) // m) * m


def _make_deg_kernel(n, e_pad):
    npad = _round_up(n, _NSUB * 8)
    rps = npad // _NSUB
    ept = e_pad // _NW
    nchunks = ept // _K
    mesh = plsc.VectorSubcoreMesh(core_axis_name="c", subcore_axis_name="s")

    @functools.partial(
        pl.kernel,
        out_type=jax.ShapeDtypeStruct((_NCORE, npad, 16), _f32),
        mesh=mesh,
        compiler_params=_SC_PARAMS,
        scratch_types=[
            pltpu.VMEM((_K,), _i32),
            pltpu.VMEM((_K,), _f32),
            pltpu.VMEM((_K, 16), _f32),
            pltpu.VMEM_SHARED((npad, 16), _f32),
        ],
    )
    def deg_kernel(d_hbm, w_hbm, out_hbm, didx_v, w_v, rows_v, acc_sh):
        cid = lax.axis_index("c")
        sid = lax.axis_index("s")
        wid = sid * _NCORE + cid

        _zero2d(rows_v, _K, 16)
        off = 0
        while off < rps:
            csz = min(_K, rps - off)
            pltpu.sync_copy(rows_v.at[pl.ds(0, csz)],
                            acc_sh.at[pl.ds(sid * rps + off, csz)])
            off += csz
        plsc.subcore_barrier()
        it = _iota16()

        def chunk(k, carry):
            base = wid * ept + k * _K
            pltpu.sync_copy(d_hbm.at[pl.ds(base, _K)], didx_v)
            pltpu.sync_copy(w_hbm.at[pl.ds(base, _K)], w_v)

            def grp(j, carry2):
                w16 = w_v[pl.ds(j * 16, 16)]
                rows16 = it + j * 16
                for c in range(16):
                    plsc.store_scatter(
                        rows_v, [rows16, jnp.full((16,), c, _i32)], w16)
                return carry2
            lax.fori_loop(0, _K // 16, grp, 0)
            pltpu.sync_copy(rows_v, acc_sh.at[didx_v], add=True)
            return carry
        lax.fori_loop(0, nchunks, chunk, 0)
        plsc.subcore_barrier()

        off = 0
        while off < rps:
            csz = min(_K, rps - off)
            pltpu.sync_copy(acc_sh.at[pl.ds(sid * rps + off, csz)],
                            out_hbm.at[cid, pl.ds(sid * rps + off, csz)])
            off += csz

    return deg_kernel


def _make_msg_kernel(n, e_pad):
    npad = _round_up(n, _NSUB * 8)
    rps = npad // _NSUB
    ept = e_pad // _NW
    nchunks = ept // _K
    mesh = plsc.VectorSubcoreMesh(core_axis_name="c", subcore_axis_name="s")

    @functools.partial(
        pl.kernel,
        out_type=jax.ShapeDtypeStruct((_NCORE, npad, _H), _f32),
        mesh=mesh,
        compiler_params=_SC_PARAMS,
        scratch_types=[
            pltpu.VMEM((_K,), _i32),
            pltpu.VMEM((_K,), _i32),
            pltpu.VMEM((_K,), _f32),
            pltpu.VMEM((_K, _H), _f32),
            pltpu.VMEM_SHARED((npad, _H), _f32),
            pltpu.VMEM_SHARED((npad, _H), _f32),
            pltpu.SemaphoreType.DMA,
        ],
    )
    def msg_kernel(g_hbm, s_hbm, d_hbm, w_hbm, out_hbm,
                   sidx_v, didx_v, w_v, rows_v, acc_sh, g_sh, sem):
        cid = lax.axis_index("c")
        sid = lax.axis_index("s")
        wid = sid * _NCORE + cid

        pltpu.sync_copy(g_hbm.at[pl.ds(sid * rps, rps)],
                        g_sh.at[pl.ds(sid * rps, rps)])

        _zero2d(rows_v, _K, _H)
        off = 0
        while off < rps:
            csz = min(_K, rps - off)
            pltpu.sync_copy(rows_v.at[pl.ds(0, csz)],
                            acc_sh.at[pl.ds(sid * rps + off, csz)])
            off += csz
        plsc.subcore_barrier()
        it = _iota16()

        def chunk(k, carry):
            base = wid * ept + k * _K
            pltpu.sync_copy(s_hbm.at[pl.ds(base, _K)], sidx_v)
            pltpu.sync_copy(d_hbm.at[pl.ds(base, _K)], didx_v)
            pltpu.sync_copy(w_hbm.at[pl.ds(base, _K)], w_v)
            pltpu.async_copy(g_sh.at[sidx_v], rows_v, sem).wait()

            def grp(j, carry2):
                w16 = w_v[pl.ds(j * 16, 16)]
                rows16 = it + j * 16
                for c in range(_H):
                    colv = jnp.full((16,), c, _i32)
                    vals = plsc.load_gather(rows_v, [rows16, colv])
                    plsc.store_scatter(rows_v, [rows16, colv], vals * w16)
                return carry2
            lax.fori_loop(0, _K // 16, grp, 0)
            pltpu.sync_copy(rows_v, acc_sh.at[didx_v], add=True)
            return carry
        lax.fori_loop(0, nchunks, chunk, 0)
        plsc.subcore_barrier()

        off = 0
        while off < rps:
            csz = min(_K, rps - off)
            pltpu.sync_copy(acc_sh.at[pl.ds(sid * rps + off, csz)],
                            out_hbm.at[cid, pl.ds(sid * rps + off, csz)])
            off += csz

    return msg_kernel


def _make_cover_kernel():
    nout = _NW * _CLPT
    mesh = plsc.VectorSubcoreMesh(core_axis_name="c", subcore_axis_name="s")

    @functools.partial(
        pl.kernel,
        out_type=(jax.ShapeDtypeStruct((nout, _H), _f32),
                  jax.ShapeDtypeStruct((nout, _H), _f32)),
        mesh=mesh,
        compiler_params=_SC_PARAMS,
        scratch_types=[
            pltpu.VMEM((_CK,), _i32),
            pltpu.VMEM((_CK,), _i32),
            pltpu.VMEM((_CK, _H), _f32),
            pltpu.VMEM((_CLPT, _H), _f32),
            pltpu.VMEM((_CLPT, _H), _f32),
            pltpu.SemaphoreType.DMA,
        ],
    )
    def cover_kernel(h_hbm, cc_hbm, cn_hbm, out_add, out_max,
                     cc_v, cn_v, rows_v, asum, amax, sem):
        cid = lax.axis_index("c")
        sid = lax.axis_index("s")
        wid = sid * _NCORE + cid
        c_lo = wid * _CLPT
        c_hi = jnp.minimum(c_lo + _CLPT, _N1)

        def cnt_chunk(k, carry):
            lo, hi = carry
            pltpu.sync_copy(cc_hbm.at[pl.ds(k * _CK, _CK)], cc_v)

            def cnt_grp(j, carry2):
                lo2, hi2 = carry2
                v = cc_v[pl.ds(j * 16, 16)]
                lo2 = lo2 + plsc.all_reduce_population_count(v < c_lo)
                hi2 = hi2 + plsc.all_reduce_population_count(v < c_hi)
                return lo2, hi2
            return lax.fori_loop(0, _CK // 16, cnt_grp, (lo, hi))
        z16 = jnp.zeros((16,), _i32)
        lo_v, hi_v = lax.fori_loop(0, _CP // _CK, cnt_chunk, (z16, z16))
        e_lo = jnp.max(lo_v)
        e_hi = jnp.max(hi_v)
        estart = (e_lo // 8) * 8
        nch = (e_hi - estart + _CK - 1) // _CK

        _zero2d(asum, _CLPT, _H)
        _zero2d(amax, _CLPT, _H)
        it = _iota16()

        def chunk(k, carry):
            base = estart + k * _CK
            pltpu.sync_copy(cc_hbm.at[pl.ds(base, _CK)], cc_v)
            pltpu.sync_copy(cn_hbm.at[pl.ds(base, _CK)], cn_v)
            pltpu.async_copy(h_hbm.at[cn_v], rows_v, sem).wait()

            def ent(j, carry2):
                ci16 = cc_v[pl.ds(j * 16, 16)]
                for i in range(16):
                    ci = ci16[i]
                    inr = jnp.logical_and(ci >= c_lo, ci < c_hi)
                    l = jnp.clip(ci - c_lo, 0, _CLPT - 1)
                    m = jnp.where(inr, jnp.float32(1.0), jnp.float32(0.0))
                    rsel = jnp.full((16,), j * 16 + i, _i32)
                    lsel = jnp.full((16,), l, _i32)
                    for c in range(_H // 16):
                        cols = it + c * 16
                        val = plsc.load_gather(rows_v, [rsel, cols]) * m
                        so = plsc.load_gather(asum, [lsel, cols])
                        plsc.store_scatter(asum, [lsel, cols], so + val)
                        mo = plsc.load_gather(amax, [lsel, cols])
                        plsc.store_scatter(amax, [lsel, cols],
                                           jnp.maximum(mo, val))
                return carry2
            lax.fori_loop(0, _CK // 16, ent, 0)
            return carry
        lax.fori_loop(0, nch, chunk, 0)

        pltpu.sync_copy(asum, out_add.at[pl.ds(c_lo, _CLPT)])
        pltpu.sync_copy(amax, out_max.at[pl.ds(c_lo, _CLPT)])

    return cover_kernel


def _mm_first_body(x_ref, w_ref, degp_ref, g_ref, dinv_ref):
    dinv = lax.rsqrt(1.0 + degp_ref[0, :, 0:1] + degp_ref[1, :, 0:1])
    g_ref[...] = dinv * jnp.dot(x_ref[...], w_ref[...],
                                preferred_element_type=_f32)
    dinv_ref[...] = dinv


def _mm_first(x, w, degp, n):
    grid = pl.cdiv(n, _R)
    return pl.pallas_call(
        _mm_first_body,
        grid=(grid,),
        in_specs=[
            pl.BlockSpec((_R, x.shape[1]), lambda i: (i, 0)),
            pl.BlockSpec(w.shape, lambda i: (0, 0)),
            pl.BlockSpec((_NCORE, _R, 16), lambda i: (0, i, 0)),
        ],
        out_specs=[
            pl.BlockSpec((_R, _H), lambda i: (i, 0)),
            pl.BlockSpec((_R, 1), lambda i: (i, 0)),
        ],
        out_shape=[
            jax.ShapeDtypeStruct((n, _H), _f32),
            jax.ShapeDtypeStruct((n, 1), _f32),
        ],
    )(x, w, degp)


def _mm_pair_body(xa_ref, xb_ref, wt_ref, wb_ref, degp_ref, g_ref, dinv_ref):
    dinv = lax.rsqrt(1.0 + degp_ref[0, :, 0:1] + degp_ref[1, :, 0:1])
    acc = (jnp.dot(xa_ref[...], wt_ref[...], preferred_element_type=_f32)
           + jnp.dot(xb_ref[...], wb_ref[...], preferred_element_type=_f32))
    g_ref[...] = dinv * acc
    dinv_ref[...] = dinv


def _mm_pair(xa, xb, wt, wb, degp, n):
    grid = pl.cdiv(n, _R)
    return pl.pallas_call(
        _mm_pair_body,
        grid=(grid,),
        in_specs=[
            pl.BlockSpec((_R, _H), lambda i: (i, 0)),
            pl.BlockSpec((_R, _H), lambda i: (i, 0)),
            pl.BlockSpec((_H, _H), lambda i: (0, 0)),
            pl.BlockSpec((_H, _H), lambda i: (0, 0)),
            pl.BlockSpec((_NCORE, _R, 16), lambda i: (0, i, 0)),
        ],
        out_specs=[
            pl.BlockSpec((_R, _H), lambda i: (i, 0)),
            pl.BlockSpec((_R, 1), lambda i: (i, 0)),
        ],
        out_shape=[
            jax.ShapeDtypeStruct((n, _H), _f32),
            jax.ShapeDtypeStruct((n, 1), _f32),
        ],
    )(xa, xb, wt, wb, degp)


def _mid_body(msgp_ref, g_ref, dinv_ref, b_ref, w_ref, h_ref, g2_ref):
    dinv = dinv_ref[...]
    h = jnp.maximum(
        dinv * (msgp_ref[0] + msgp_ref[1] + g_ref[...]) + b_ref[...], 0.0)
    h_ref[...] = h
    g2_ref[...] = dinv * jnp.dot(h, w_ref[...], preferred_element_type=_f32)


def _mid(msgp, g, dinv, b, w, n):
    grid = pl.cdiv(n, _R)
    return pl.pallas_call(
        _mid_body,
        grid=(grid,),
        in_specs=[
            pl.BlockSpec((_NCORE, _R, _H), lambda i: (0, i, 0)),
            pl.BlockSpec((_R, _H), lambda i: (i, 0)),
            pl.BlockSpec((_R, 1), lambda i: (i, 0)),
            pl.BlockSpec((1, _H), lambda i: (0, 0)),
            pl.BlockSpec((_H, _H), lambda i: (0, 0)),
        ],
        out_specs=[
            pl.BlockSpec((_R, _H), lambda i: (i, 0)),
            pl.BlockSpec((_R, _H), lambda i: (i, 0)),
        ],
        out_shape=[
            jax.ShapeDtypeStruct((n, _H), _f32),
            jax.ShapeDtypeStruct((n, _H), _f32),
        ],
    )(msgp, g, dinv, b, w)


def _blockout_body(msgp_ref, g2_ref, dinv_ref, b2_ref, h1_ref,
                   wlt_ref, wlb_ref, bl_ref, h_ref):
    dinv = dinv_ref[...]
    h2 = jnp.maximum(
        dinv * (msgp_ref[0] + msgp_ref[1] + g2_ref[...]) + b2_ref[...], 0.0)
    acc = (jnp.dot(h1_ref[...], wlt_ref[...], preferred_element_type=_f32)
           + jnp.dot(h2, wlb_ref[...], preferred_element_type=_f32))
    h_ref[...] = jnp.maximum(acc + bl_ref[...], 0.0)


def _blockout(msgp, g2, dinv, b2, h1, wlt, wlb, bl, n):
    grid = pl.cdiv(n, _R)
    return pl.pallas_call(
        _blockout_body,
        grid=(grid,),
        in_specs=[
            pl.BlockSpec((_NCORE, _R, _H), lambda i: (0, i, 0)),
            pl.BlockSpec((_R, _H), lambda i: (i, 0)),
            pl.BlockSpec((_R, 1), lambda i: (i, 0)),
            pl.BlockSpec((1, _H), lambda i: (0, 0)),
            pl.BlockSpec((_R, _H), lambda i: (i, 0)),
            pl.BlockSpec((_H, _H), lambda i: (0, 0)),
            pl.BlockSpec((_H, _H), lambda i: (0, 0)),
            pl.BlockSpec((1, _H), lambda i: (0, 0)),
        ],
        out_specs=pl.BlockSpec((_R, _H), lambda i: (i, 0)),
        out_shape=jax.ShapeDtypeStruct((n, _H), _f32),
    )(msgp, g2, dinv, b2, h1, wlt, wlb, bl)


def _pool_body(n, h_ref, bidx_ref, sum_ref, max_ref):
    i = pl.program_id(0)
    h = h_ref[...]
    bid = bidx_ref[...]
    row = i * _R + lax.broadcasted_iota(_i32, (_R, 1), 0)
    valid = row < n
    onehot = jnp.logical_and(
        bid == lax.broadcasted_iota(_i32, (1, _B), 1), valid).astype(_f32)
    s = lax.dot_general(onehot, h, (((0,), (0,)), ((), ())),
                        preferred_element_type=_f32)
    rows = []
    for b in range(_B):
        m = jnp.where(jnp.logical_and(bid == b, valid), h, 0.0)
        rows.append(jnp.max(m, axis=0, keepdims=True))
    mx = jnp.concatenate(rows, axis=0)

    @pl.when(i == 0)
    def _():
        sum_ref[...] = s
        max_ref[...] = mx

    @pl.when(i > 0)
    def _():
        sum_ref[...] = sum_ref[...] + s
        max_ref[...] = jnp.maximum(max_ref[...], mx)


def _pool(h, bidx, n):
    grid = pl.cdiv(n, _R)
    return pl.pallas_call(
        functools.partial(_pool_body, n),
        grid=(grid,),
        in_specs=[
            pl.BlockSpec((_R, _H), lambda i: (i, 0)),
            pl.BlockSpec((_R, 1), lambda i: (i, 0)),
        ],
        out_specs=[
            pl.BlockSpec((_B, _H), lambda i: (0, 0)),
            pl.BlockSpec((_B, _H), lambda i: (0, 0)),
        ],
        out_shape=[
            jax.ShapeDtypeStruct((_B, _H), _f32),
            jax.ShapeDtypeStruct((_B, _H), _f32),
        ],
    )(h, bidx)


def _head_body(x0, x1, x2, x3, gamma, beta, w1, b1, w2, b2, out):
    z = jnp.concatenate([x0[...], x1[...], x2[...], x3[...]], axis=1)
    mu = jnp.mean(z, axis=0, keepdims=True)
    var = jnp.mean((z - mu) ** 2, axis=0, keepdims=True)
    z = (z - mu) * lax.rsqrt(var + 1e-5) * gamma[...] + beta[...]
    z = jnp.maximum(jnp.dot(z, w1[...], preferred_element_type=_f32)
                    + b1[...], 0.0)
    z = jnp.dot(z, w2[...], preferred_element_type=_f32) + b2[...]
    z = z - jnp.max(z, axis=1, keepdims=True)
    ez = jnp.exp(z)
    out[...] = ez / jnp.sum(ez, axis=1, keepdims=True)


def _head(x0, x1, x2, x3, gamma, beta, w1, b1, w2, b2):
    return pl.pallas_call(
        _head_body,
        out_shape=jax.ShapeDtypeStruct((_B, _NC), _f32),
    )(x0, x1, x2, x3, gamma, beta, w1, b1, w2, b2)


_deg0 = _make_deg_kernel(_N0, _EP0)
_deg1 = _make_deg_kernel(_N1, _EP1)
_msg0 = _make_msg_kernel(_N0, _EP0)
_msg1 = _make_msg_kernel(_N1, _EP1)
_cover = _make_cover_kernel()


def kernel(x, edge_index, edge_weight, batch, cover_node, cover_cluster,
           edge_index2, edge_weight2, batch2, cW1, cb1, cW2, cb2, cWl, cbl,
           bW1, bb1, bW2, bb2, bWl, bbl, gamma, beta, l1W, l1b, l2W, l2b):
    s0 = jnp.pad(edge_index[0], (0, _EP0 - _E0))
    d0 = jnp.pad(edge_index[1], (0, _EP0 - _E0))
    w0 = jnp.pad(edge_weight, (0, _EP0 - _E0))
    s1 = jnp.pad(edge_index2[0], (0, _EP1 - _E1))
    d1 = jnp.pad(edge_index2[1], (0, _EP1 - _E1))
    w1 = jnp.pad(edge_weight2, (0, _EP1 - _E1))
    ccp = jnp.pad(cover_cluster, (0, _CP - _C), constant_values=1 << 30)
    cnp = jnp.pad(cover_node, (0, _CP - _C))

    degp0 = _deg0(d0, w0)
    degp1 = _deg1(d1, w1)

    g1, dinv0 = _mm_first(x, cW1, degp0, _N0)
    mp = _msg0(jnp.pad(g1, ((0, _NP0 - _N0), (0, 0))), s0, d0, w0)
    h1, g2 = _mid(mp, g1, dinv0, cb1.reshape(1, _H), cW2, _N0)
    mp = _msg0(jnp.pad(g2, ((0, _NP0 - _N0), (0, 0))), s0, d0, w0)
    h = _blockout(mp, g2, dinv0, cb2.reshape(1, _H), h1,
                  cWl[:_H], cWl[_H:], cbl.reshape(1, _H), _N0)

    xs0, xs1 = _pool(h, batch.reshape(_N0, 1), _N0)
    xadd_p, xmax_p = _cover(h, ccp, cnp)
    x_add = xadd_p[:_N1]
    x_max = xmax_p[:_N1]

    gB, dinv1 = _mm_pair(x_add, x_max, bW1[:_H], bW1[_H:], degp1, _N1)
    mp = _msg1(jnp.pad(gB, ((0, _NP1 - _N1), (0, 0))), s1, d1, w1)
    h1B, g2B = _mid(mp, gB, dinv1, bb1.reshape(1, _H), bW2, _N1)
    mp = _msg1(jnp.pad(g2B, ((0, _NP1 - _N1), (0, 0))), s1, d1, w1)
    hB = _blockout(mp, g2B, dinv1, bb2.reshape(1, _H), h1B,
                   bWl[:_H], bWl[_H:], bbl.reshape(1, _H), _N1)

    xs2, xs3 = _pool(hB, batch2.reshape(_N1, 1), _N1)

    return _head(xs0, xs1, xs2, xs3,
                 gamma.reshape(1, 4 * _H), beta.reshape(1, 4 * _H),
                 l1W, l1b.reshape(1, _H), l2W, l2b.reshape(1, _NC))


# speedup vs baseline: 19.3383x; 3.9607x over previous
import functools

import jax
import jax.numpy as jnp
from jax import lax
from jax.experimental import pallas as pl
from jax.experimental.pallas import tpu as pltpu
from jax.experimental.pallas import tpu_sc as plsc

_N0, _E0, _D, _H, _B = 10000, 320000, 128, 64, 16
_C, _N1, _E1, _NC = 15000, 2500, 80000, 10
_NCORE, _NSUB, _NW = 2, 16, 32
_K = 512
_KM = 128
_CK = 256
_CLPT = 80
_R = 512

_EP0 = 327680
_EP1 = 81920
_CP = 15360
_NP0 = 10112
_NP1 = 2560

_f32 = jnp.float32
_i32 = jnp.int32


def _iota16():
    return lax.iota(_i32, 16)


def _zero2d(ref, nrows, ncols):
    z = jnp.zeros((16,), _f32)
    it = _iota16()

    def zrow(j, carry):
        rows16 = it + j * 16
        for c in range(ncols):
            plsc.store_scatter(ref, [rows16, jnp.full((16,), c, _i32)], z)
        return carry
    lax.fori_loop(0, nrows // 16, zrow, 0)


_SC_PARAMS = pltpu.CompilerParams(needs_layout_passes=False,
                                  use_tc_tiling_on_sc=False)


def _round_up(a, m):
    return ((a + m - 1) // m) * m


def _make_deg_kernel(n, e_pad):
    npad = _round_up(n, _NSUB * 8)
    rps = npad // _NSUB
    ept = e_pad // _NW
    nchunks = ept // _K
    mesh = plsc.VectorSubcoreMesh(core_axis_name="c", subcore_axis_name="s")

    @functools.partial(
        pl.kernel,
        out_type=jax.ShapeDtypeStruct((_NCORE, npad, 16), _f32),
        mesh=mesh,
        compiler_params=_SC_PARAMS,
        scratch_types=[
            pltpu.VMEM((_K,), _i32),
            pltpu.VMEM((_K,), _f32),
            pltpu.VMEM((_K, 16), _f32),
            pltpu.VMEM_SHARED((npad, 16), _f32),
        ],
    )
    def deg_kernel(d_hbm, w_hbm, out_hbm, didx_v, w_v, rows_v, acc_sh):
        cid = lax.axis_index("c")
        sid = lax.axis_index("s")
        wid = sid * _NCORE + cid

        _zero2d(rows_v, _K, 16)
        off = 0
        while off < rps:
            csz = min(_K, rps - off)
            pltpu.sync_copy(rows_v.at[pl.ds(0, csz)],
                            acc_sh.at[pl.ds(sid * rps + off, csz)])
            off += csz
        plsc.subcore_barrier()
        it = _iota16()

        def chunk(k, carry):
            base = wid * ept + k * _K
            pltpu.sync_copy(d_hbm.at[pl.ds(base, _K)], didx_v)
            pltpu.sync_copy(w_hbm.at[pl.ds(base, _K)], w_v)

            def grp(j, carry2):
                w16 = w_v[pl.ds(j * 16, 16)]
                rows16 = it + j * 16
                for c in range(16):
                    plsc.store_scatter(
                        rows_v, [rows16, jnp.full((16,), c, _i32)], w16)
                return carry2
            lax.fori_loop(0, _K // 16, grp, 0)
            pltpu.sync_copy(rows_v, acc_sh.at[didx_v], add=True)
            return carry
        lax.fori_loop(0, nchunks, chunk, 0)
        plsc.subcore_barrier()

        off = 0
        while off < rps:
            csz = min(_K, rps - off)
            pltpu.sync_copy(acc_sh.at[pl.ds(sid * rps + off, csz)],
                            out_hbm.at[cid, pl.ds(sid * rps + off, csz)])
            off += csz

    return deg_kernel


def _make_msg_kernel(n, e_pad):
    npad = _round_up(n, _NSUB * 8)
    rps = npad // _NSUB
    ept = e_pad // _NW
    nchunks = ept // _KM
    mesh = plsc.VectorSubcoreMesh(core_axis_name="c", subcore_axis_name="s")

    @functools.partial(
        pl.kernel,
        out_type=jax.ShapeDtypeStruct((_NCORE, npad, _H), _f32),
        mesh=mesh,
        compiler_params=_SC_PARAMS,
        scratch_types=[
            pltpu.VMEM((ept,), _i32),
            pltpu.VMEM((ept,), _i32),
            pltpu.VMEM((ept,), _f32),
            pltpu.VMEM((_KM, _H), _f32),
            pltpu.VMEM((_KM, _H), _f32),
            pltpu.VMEM_SHARED((npad, _H), _f32),
            pltpu.VMEM_SHARED((npad, _H), _f32),
            pltpu.SemaphoreType.DMA,
            pltpu.SemaphoreType.DMA,
            pltpu.SemaphoreType.DMA,
            pltpu.SemaphoreType.DMA,
        ],
    )
    def msg_kernel(g_hbm, s_hbm, d_hbm, w_hbm, out_hbm,
                   sidx_v, didx_v, w_v, buf0, buf1, acc_sh, g_sh,
                   gsem0, gsem1, asem0, asem1):
        cid = lax.axis_index("c")
        sid = lax.axis_index("s")
        wid = sid * _NCORE + cid
        bufs = (buf0, buf1)
        gsem = (gsem0, gsem1)
        asem = (asem0, asem1)

        pltpu.sync_copy(g_hbm.at[pl.ds(sid * rps, rps)],
                        g_sh.at[pl.ds(sid * rps, rps)])

        _zero2d(buf0, _KM, _H)
        off = 0
        while off < rps:
            csz = min(_KM, rps - off)
            pltpu.sync_copy(buf0.at[pl.ds(0, csz)],
                            acc_sh.at[pl.ds(sid * rps + off, csz)])
            off += csz
        plsc.subcore_barrier()

        ebase = wid * ept
        pltpu.sync_copy(s_hbm.at[pl.ds(ebase, ept)], sidx_v)
        pltpu.sync_copy(d_hbm.at[pl.ds(ebase, ept)], didx_v)
        pltpu.sync_copy(w_hbm.at[pl.ds(ebase, ept)], w_v)

        def gather(c, b):
            return pltpu.async_copy(
                g_sh.at[sidx_v.at[pl.ds(c * _KM, _KM)]], bufs[b], gsem[b])

        def scat_add(c, b):
            return pltpu.async_copy(
                bufs[b], acc_sh.at[didx_v.at[pl.ds(c * _KM, _KM)]], asem[b],
                add=True)

        def scale(c, b):
            buf = bufs[b]

            def grp(j, carry):
                w16 = w_v[pl.ds(c * _KM + j * 16, 16)]
                for i in range(16):
                    wsp = jnp.full((16,), w16[i], _f32)
                    row = buf.at[j * 16 + i]
                    for col in range(_H // 16):
                        v = row[pl.ds(col * 16, 16)]
                        row[pl.ds(col * 16, 16)] = v * wsp
                return carry
            lax.fori_loop(0, _KM // 16, grp, 0)

        def pair(k, carry):
            c0 = 2 * k
            h0 = gather(c0, 0)
            h1 = gather(c0 + 1, 1)
            h0.wait()
            scale(c0, 0)
            a0 = scat_add(c0, 0)
            h1.wait()
            scale(c0 + 1, 1)
            a1 = scat_add(c0 + 1, 1)
            a0.wait()
            a1.wait()
            return carry
        lax.fori_loop(0, nchunks // 2, pair, 0)
        plsc.subcore_barrier()

        off = 0
        while off < rps:
            csz = min(_KM, rps - off)
            pltpu.sync_copy(acc_sh.at[pl.ds(sid * rps + off, csz)],
                            out_hbm.at[cid, pl.ds(sid * rps + off, csz)])
            off += csz

    return msg_kernel


def _make_cover_kernel():
    nout = _NW * _CLPT
    mesh = plsc.VectorSubcoreMesh(core_axis_name="c", subcore_axis_name="s")

    @functools.partial(
        pl.kernel,
        out_type=(jax.ShapeDtypeStruct((nout, _H), _f32),
                  jax.ShapeDtypeStruct((nout, _H), _f32)),
        mesh=mesh,
        compiler_params=_SC_PARAMS,
        scratch_types=[
            pltpu.VMEM((_CK,), _i32),
            pltpu.VMEM((_CK,), _i32),
            pltpu.VMEM((_CK, _H), _f32),
            pltpu.VMEM((_CLPT, _H), _f32),
            pltpu.VMEM((_CLPT, _H), _f32),
            pltpu.SemaphoreType.DMA,
        ],
    )
    def cover_kernel(h_hbm, cc_hbm, cn_hbm, out_add, out_max,
                     cc_v, cn_v, rows_v, asum, amax, sem):
        cid = lax.axis_index("c")
        sid = lax.axis_index("s")
        wid = sid * _NCORE + cid
        c_lo = wid * _CLPT
        c_hi = jnp.minimum(c_lo + _CLPT, _N1)

        def cnt_chunk(k, carry):
            lo, hi = carry
            pltpu.sync_copy(cc_hbm.at[pl.ds(k * _CK, _CK)], cc_v)

            def cnt_grp(j, carry2):
                lo2, hi2 = carry2
                v = cc_v[pl.ds(j * 16, 16)]
                lo2 = lo2 + plsc.all_reduce_population_count(v < c_lo)
                hi2 = hi2 + plsc.all_reduce_population_count(v < c_hi)
                return lo2, hi2
            return lax.fori_loop(0, _CK // 16, cnt_grp, (lo, hi))
        z16 = jnp.zeros((16,), _i32)
        lo_v, hi_v = lax.fori_loop(0, _CP // _CK, cnt_chunk, (z16, z16))
        e_lo = jnp.max(lo_v)
        e_hi = jnp.max(hi_v)
        estart = (e_lo // 8) * 8
        nch = (e_hi - estart + _CK - 1) // _CK

        _zero2d(asum, _CLPT, _H)
        _zero2d(amax, _CLPT, _H)
        it = _iota16()

        def chunk(k, carry):
            base = estart + k * _CK
            pltpu.sync_copy(cc_hbm.at[pl.ds(base, _CK)], cc_v)
            pltpu.sync_copy(cn_hbm.at[pl.ds(base, _CK)], cn_v)
            pltpu.async_copy(h_hbm.at[cn_v], rows_v, sem).wait()

            def ent(j, carry2):
                ci16 = cc_v[pl.ds(j * 16, 16)]
                for i in range(16):
                    ci = ci16[i]
                    inr = jnp.logical_and(ci >= c_lo, ci < c_hi)
                    l = jnp.clip(ci - c_lo, 0, _CLPT - 1)
                    m = jnp.where(inr, jnp.float32(1.0), jnp.float32(0.0))
                    rsel = jnp.full((16,), j * 16 + i, _i32)
                    lsel = jnp.full((16,), l, _i32)
                    for c in range(_H // 16):
                        cols = it + c * 16
                        val = plsc.load_gather(rows_v, [rsel, cols]) * m
                        so = plsc.load_gather(asum, [lsel, cols])
                        plsc.store_scatter(asum, [lsel, cols], so + val)
                        mo = plsc.load_gather(amax, [lsel, cols])
                        plsc.store_scatter(amax, [lsel, cols],
                                           jnp.maximum(mo, val))
                return carry2
            lax.fori_loop(0, _CK // 16, ent, 0)
            return carry
        lax.fori_loop(0, nch, chunk, 0)

        pltpu.sync_copy(asum, out_add.at[pl.ds(c_lo, _CLPT)])
        pltpu.sync_copy(amax, out_max.at[pl.ds(c_lo, _CLPT)])

    return cover_kernel


def _mm_first_body(x_ref, w_ref, degp_ref, g_ref, dinv_ref):
    dinv = lax.rsqrt(1.0 + degp_ref[0, :, 0:1] + degp_ref[1, :, 0:1])
    g_ref[...] = dinv * jnp.dot(x_ref[...], w_ref[...],
                                preferred_element_type=_f32)
    dinv_ref[...] = dinv


def _mm_first(x, w, degp, n):
    grid = pl.cdiv(n, _R)
    return pl.pallas_call(
        _mm_first_body,
        grid=(grid,),
        in_specs=[
            pl.BlockSpec((_R, x.shape[1]), lambda i: (i, 0)),
            pl.BlockSpec(w.shape, lambda i: (0, 0)),
            pl.BlockSpec((_NCORE, _R, 16), lambda i: (0, i, 0)),
        ],
        out_specs=[
            pl.BlockSpec((_R, _H), lambda i: (i, 0)),
            pl.BlockSpec((_R, 1), lambda i: (i, 0)),
        ],
        out_shape=[
            jax.ShapeDtypeStruct((n, _H), _f32),
            jax.ShapeDtypeStruct((n, 1), _f32),
        ],
    )(x, w, degp)


def _mm_pair_body(xa_ref, xb_ref, wt_ref, wb_ref, degp_ref, g_ref, dinv_ref):
    dinv = lax.rsqrt(1.0 + degp_ref[0, :, 0:1] + degp_ref[1, :, 0:1])
    acc = (jnp.dot(xa_ref[...], wt_ref[...], preferred_element_type=_f32)
           + jnp.dot(xb_ref[...], wb_ref[...], preferred_element_type=_f32))
    g_ref[...] = dinv * acc
    dinv_ref[...] = dinv


def _mm_pair(xa, xb, wt, wb, degp, n):
    grid = pl.cdiv(n, _R)
    return pl.pallas_call(
        _mm_pair_body,
        grid=(grid,),
        in_specs=[
            pl.BlockSpec((_R, _H), lambda i: (i, 0)),
            pl.BlockSpec((_R, _H), lambda i: (i, 0)),
            pl.BlockSpec((_H, _H), lambda i: (0, 0)),
            pl.BlockSpec((_H, _H), lambda i: (0, 0)),
            pl.BlockSpec((_NCORE, _R, 16), lambda i: (0, i, 0)),
        ],
        out_specs=[
            pl.BlockSpec((_R, _H), lambda i: (i, 0)),
            pl.BlockSpec((_R, 1), lambda i: (i, 0)),
        ],
        out_shape=[
            jax.ShapeDtypeStruct((n, _H), _f32),
            jax.ShapeDtypeStruct((n, 1), _f32),
        ],
    )(xa, xb, wt, wb, degp)


def _mid_body(msgp_ref, g_ref, dinv_ref, b_ref, w_ref, h_ref, g2_ref):
    dinv = dinv_ref[...]
    h = jnp.maximum(
        dinv * (msgp_ref[0] + msgp_ref[1] + g_ref[...]) + b_ref[...], 0.0)
    h_ref[...] = h
    g2_ref[...] = dinv * jnp.dot(h, w_ref[...], preferred_element_type=_f32)


def _mid(msgp, g, dinv, b, w, n):
    grid = pl.cdiv(n, _R)
    return pl.pallas_call(
        _mid_body,
        grid=(grid,),
        in_specs=[
            pl.BlockSpec((_NCORE, _R, _H), lambda i: (0, i, 0)),
            pl.BlockSpec((_R, _H), lambda i: (i, 0)),
            pl.BlockSpec((_R, 1), lambda i: (i, 0)),
            pl.BlockSpec((1, _H), lambda i: (0, 0)),
            pl.BlockSpec((_H, _H), lambda i: (0, 0)),
        ],
        out_specs=[
            pl.BlockSpec((_R, _H), lambda i: (i, 0)),
            pl.BlockSpec((_R, _H), lambda i: (i, 0)),
        ],
        out_shape=[
            jax.ShapeDtypeStruct((n, _H), _f32),
            jax.ShapeDtypeStruct((n, _H), _f32),
        ],
    )(msgp, g, dinv, b, w)


def _blockout_body(msgp_ref, g2_ref, dinv_ref, b2_ref, h1_ref,
                   wlt_ref, wlb_ref, bl_ref, h_ref):
    dinv = dinv_ref[...]
    h2 = jnp.maximum(
        dinv * (msgp_ref[0] + msgp_ref[1] + g2_ref[...]) + b2_ref[...], 0.0)
    acc = (jnp.dot(h1_ref[...], wlt_ref[...], preferred_element_type=_f32)
           + jnp.dot(h2, wlb_ref[...], preferred_element_type=_f32))
    h_ref[...] = jnp.maximum(acc + bl_ref[...], 0.0)


def _blockout(msgp, g2, dinv, b2, h1, wlt, wlb, bl, n):
    grid = pl.cdiv(n, _R)
    return pl.pallas_call(
        _blockout_body,
        grid=(grid,),
        in_specs=[
            pl.BlockSpec((_NCORE, _R, _H), lambda i: (0, i, 0)),
            pl.BlockSpec((_R, _H), lambda i: (i, 0)),
            pl.BlockSpec((_R, 1), lambda i: (i, 0)),
            pl.BlockSpec((1, _H), lambda i: (0, 0)),
            pl.BlockSpec((_R, _H), lambda i: (i, 0)),
            pl.BlockSpec((_H, _H), lambda i: (0, 0)),
            pl.BlockSpec((_H, _H), lambda i: (0, 0)),
            pl.BlockSpec((1, _H), lambda i: (0, 0)),
        ],
        out_specs=pl.BlockSpec((_R, _H), lambda i: (i, 0)),
        out_shape=jax.ShapeDtypeStruct((n, _H), _f32),
    )(msgp, g2, dinv, b2, h1, wlt, wlb, bl)


def _pool_body(n, h_ref, bidx_ref, sum_ref, max_ref):
    i = pl.program_id(0)
    h = h_ref[...]
    bid = bidx_ref[...]
    row = i * _R + lax.broadcasted_iota(_i32, (_R, 1), 0)
    valid = row < n
    onehot = jnp.logical_and(
        bid == lax.broadcasted_iota(_i32, (1, _B), 1), valid).astype(_f32)
    s = lax.dot_general(onehot, h, (((0,), (0,)), ((), ())),
                        preferred_element_type=_f32)
    rows = []
    for b in range(_B):
        m = jnp.where(jnp.logical_and(bid == b, valid), h, 0.0)
        rows.append(jnp.max(m, axis=0, keepdims=True))
    mx = jnp.concatenate(rows, axis=0)

    @pl.when(i == 0)
    def _():
        sum_ref[...] = s
        max_ref[...] = mx

    @pl.when(i > 0)
    def _():
        sum_ref[...] = sum_ref[...] + s
        max_ref[...] = jnp.maximum(max_ref[...], mx)


def _pool(h, bidx, n):
    grid = pl.cdiv(n, _R)
    return pl.pallas_call(
        functools.partial(_pool_body, n),
        grid=(grid,),
        in_specs=[
            pl.BlockSpec((_R, _H), lambda i: (i, 0)),
            pl.BlockSpec((_R, 1), lambda i: (i, 0)),
        ],
        out_specs=[
            pl.BlockSpec((_B, _H), lambda i: (0, 0)),
            pl.BlockSpec((_B, _H), lambda i: (0, 0)),
        ],
        out_shape=[
            jax.ShapeDtypeStruct((_B, _H), _f32),
            jax.ShapeDtypeStruct((_B, _H), _f32),
        ],
    )(h, bidx)


def _head_body(x0, x1, x2, x3, gamma, beta, w1, b1, w2, b2, out):
    z = jnp.concatenate([x0[...], x1[...], x2[...], x3[...]], axis=1)
    mu = jnp.mean(z, axis=0, keepdims=True)
    var = jnp.mean((z - mu) ** 2, axis=0, keepdims=True)
    z = (z - mu) * lax.rsqrt(var + 1e-5) * gamma[...] + beta[...]
    z = jnp.maximum(jnp.dot(z, w1[...], preferred_element_type=_f32)
                    + b1[...], 0.0)
    z = jnp.dot(z, w2[...], preferred_element_type=_f32) + b2[...]
    z = z - jnp.max(z, axis=1, keepdims=True)
    ez = jnp.exp(z)
    out[...] = ez / jnp.sum(ez, axis=1, keepdims=True)


def _head(x0, x1, x2, x3, gamma, beta, w1, b1, w2, b2):
    return pl.pallas_call(
        _head_body,
        out_shape=jax.ShapeDtypeStruct((_B, _NC), _f32),
    )(x0, x1, x2, x3, gamma, beta, w1, b1, w2, b2)


_deg0 = _make_deg_kernel(_N0, _EP0)
_deg1 = _make_deg_kernel(_N1, _EP1)
_msg0 = _make_msg_kernel(_N0, _EP0)
_msg1 = _make_msg_kernel(_N1, _EP1)
_cover = _make_cover_kernel()


def kernel(x, edge_index, edge_weight, batch, cover_node, cover_cluster,
           edge_index2, edge_weight2, batch2, cW1, cb1, cW2, cb2, cWl, cbl,
           bW1, bb1, bW2, bb2, bWl, bbl, gamma, beta, l1W, l1b, l2W, l2b):
    s0 = jnp.pad(edge_index[0], (0, _EP0 - _E0))
    d0 = jnp.pad(edge_index[1], (0, _EP0 - _E0))
    w0 = jnp.pad(edge_weight, (0, _EP0 - _E0))
    s1 = jnp.pad(edge_index2[0], (0, _EP1 - _E1))
    d1 = jnp.pad(edge_index2[1], (0, _EP1 - _E1))
    w1 = jnp.pad(edge_weight2, (0, _EP1 - _E1))
    ccp = jnp.pad(cover_cluster, (0, _CP - _C), constant_values=1 << 30)
    cnp = jnp.pad(cover_node, (0, _CP - _C))

    degp0 = _deg0(d0, w0)
    degp1 = _deg1(d1, w1)

    g1, dinv0 = _mm_first(x, cW1, degp0, _N0)
    mp = _msg0(jnp.pad(g1, ((0, _NP0 - _N0), (0, 0))), s0, d0, w0)
    h1, g2 = _mid(mp, g1, dinv0, cb1.reshape(1, _H), cW2, _N0)
    mp = _msg0(jnp.pad(g2, ((0, _NP0 - _N0), (0, 0))), s0, d0, w0)
    h = _blockout(mp, g2, dinv0, cb2.reshape(1, _H), h1,
                  cWl[:_H], cWl[_H:], cbl.reshape(1, _H), _N0)

    xs0, xs1 = _pool(h, batch.reshape(_N0, 1), _N0)
    xadd_p, xmax_p = _cover(h, ccp, cnp)
    x_add = xadd_p[:_N1]
    x_max = xmax_p[:_N1]

    gB, dinv1 = _mm_pair(x_add, x_max, bW1[:_H], bW1[_H:], degp1, _N1)
    mp = _msg1(jnp.pad(gB, ((0, _NP1 - _N1), (0, 0))), s1, d1, w1)
    h1B, g2B = _mid(mp, gB, dinv1, bb1.reshape(1, _H), bW2, _N1)
    mp = _msg1(jnp.pad(g2B, ((0, _NP1 - _N1), (0, 0))), s1, d1, w1)
    hB = _blockout(mp, g2B, dinv1, bb2.reshape(1, _H), h1B,
                   bWl[:_H], bWl[_H:], bbl.reshape(1, _H), _N1)

    xs2, xs3 = _pool(hB, batch2.reshape(_N1, 1), _N1)

    return _head(xs0, xs1, xs2, xs3,
                 gamma.reshape(1, 4 * _H), beta.reshape(1, 4 * _H),
                 l1W, l1b.reshape(1, _H), l2W, l2b.reshape(1, _NC))


# speedup vs baseline: 19.7872x; 1.0232x over previous
import functools

import jax
import jax.numpy as jnp
from jax import lax
from jax.experimental import pallas as pl
from jax.experimental.pallas import tpu as pltpu
from jax.experimental.pallas import tpu_sc as plsc

_N0, _E0, _D, _H, _B = 10000, 320000, 128, 64, 16
_C, _N1, _E1, _NC = 15000, 2500, 80000, 10
_NCORE, _NSUB, _NW = 2, 16, 32
_K = 512
_KM = 128
_CK = 256
_CLPT = 80
_R = 512

_EP0 = 327680
_EP1 = 81920
_CP = 15360
_NP0 = 10112
_NP1 = 2560

_f32 = jnp.float32
_i32 = jnp.int32


def _iota16():
    return lax.iota(_i32, 16)


def _zero2d(ref, nrows, ncols):
    z = jnp.zeros((16,), _f32)
    it = _iota16()

    def zrow(j, carry):
        rows16 = it + j * 16
        for c in range(ncols):
            plsc.store_scatter(ref, [rows16, jnp.full((16,), c, _i32)], z)
        return carry
    lax.fori_loop(0, nrows // 16, zrow, 0)


_SC_PARAMS = pltpu.CompilerParams(needs_layout_passes=False,
                                  use_tc_tiling_on_sc=False)


def _round_up(a, m):
    return ((a + m - 1) // m) * m


def _make_deg_kernel(n, e_pad):
    npad = _round_up(n, _NSUB * 8)
    rps = npad // _NSUB
    ept = e_pad // _NW
    nchunks = ept // _K
    mesh = plsc.VectorSubcoreMesh(core_axis_name="c", subcore_axis_name="s")

    @functools.partial(
        pl.kernel,
        out_type=jax.ShapeDtypeStruct((_NCORE, npad, 16), _f32),
        mesh=mesh,
        compiler_params=_SC_PARAMS,
        scratch_types=[
            pltpu.VMEM((ept,), _i32),
            pltpu.VMEM((ept,), _f32),
            pltpu.VMEM((_K, 16), _f32),
            pltpu.VMEM_SHARED((npad, 16), _f32),
        ],
    )
    def deg_kernel(d_hbm, w_hbm, out_hbm, didx_v, w_v, rows_v, acc_sh):
        cid = lax.axis_index("c")
        sid = lax.axis_index("s")
        wid = sid * _NCORE + cid

        _zero2d(rows_v, _K, 16)
        off = 0
        while off < rps:
            csz = min(_K, rps - off)
            pltpu.sync_copy(rows_v.at[pl.ds(0, csz)],
                            acc_sh.at[pl.ds(sid * rps + off, csz)])
            off += csz
        plsc.subcore_barrier()

        ebase = wid * ept
        pltpu.sync_copy(d_hbm.at[pl.ds(ebase, ept)], didx_v)
        pltpu.sync_copy(w_hbm.at[pl.ds(ebase, ept)], w_v)

        def chunk(k, carry):
            def grp(j, carry2):
                w16 = w_v[pl.ds(k * _K + j * 16, 16)]
                for i in range(16):
                    row = rows_v.at[j * 16 + i]
                    row[pl.ds(0, 16)] = jnp.full((16,), w16[i], _f32)
                return carry2
            lax.fori_loop(0, _K // 16, grp, 0)
            pltpu.sync_copy(rows_v,
                            acc_sh.at[didx_v.at[pl.ds(k * _K, _K)]], add=True)
            return carry
        lax.fori_loop(0, nchunks, chunk, 0)
        plsc.subcore_barrier()

        off = 0
        while off < rps:
            csz = min(_K, rps - off)
            pltpu.sync_copy(acc_sh.at[pl.ds(sid * rps + off, csz)],
                            out_hbm.at[cid, pl.ds(sid * rps + off, csz)])
            off += csz

    return deg_kernel


def _make_msg_kernel(n, e_pad):
    npad = _round_up(n, _NSUB * 8)
    rps = npad // _NSUB
    ept = e_pad // _NW
    nchunks = ept // _KM
    mesh = plsc.VectorSubcoreMesh(core_axis_name="c", subcore_axis_name="s")

    @functools.partial(
        pl.kernel,
        out_type=jax.ShapeDtypeStruct((_NCORE, npad, _H), _f32),
        mesh=mesh,
        compiler_params=_SC_PARAMS,
        scratch_types=[
            pltpu.VMEM((ept,), _i32),
            pltpu.VMEM((ept,), _i32),
            pltpu.VMEM((ept,), _f32),
            pltpu.VMEM((_KM, _H), _f32),
            pltpu.VMEM((_KM, _H), _f32),
            pltpu.VMEM_SHARED((npad, _H), _f32),
            pltpu.VMEM_SHARED((npad, _H), _f32),
            pltpu.SemaphoreType.DMA,
            pltpu.SemaphoreType.DMA,
            pltpu.SemaphoreType.DMA,
            pltpu.SemaphoreType.DMA,
        ],
    )
    def msg_kernel(g_hbm, s_hbm, d_hbm, w_hbm, out_hbm,
                   sidx_v, didx_v, w_v, buf0, buf1, acc_sh, g_sh,
                   gsem0, gsem1, asem0, asem1):
        cid = lax.axis_index("c")
        sid = lax.axis_index("s")
        wid = sid * _NCORE + cid
        bufs = (buf0, buf1)
        gsem = (gsem0, gsem1)
        asem = (asem0, asem1)

        pltpu.sync_copy(g_hbm.at[pl.ds(sid * rps, rps)],
                        g_sh.at[pl.ds(sid * rps, rps)])

        _zero2d(buf0, _KM, _H)
        off = 0
        while off < rps:
            csz = min(_KM, rps - off)
            pltpu.sync_copy(buf0.at[pl.ds(0, csz)],
                            acc_sh.at[pl.ds(sid * rps + off, csz)])
            off += csz
        plsc.subcore_barrier()

        ebase = wid * ept
        pltpu.sync_copy(s_hbm.at[pl.ds(ebase, ept)], sidx_v)
        pltpu.sync_copy(d_hbm.at[pl.ds(ebase, ept)], didx_v)
        pltpu.sync_copy(w_hbm.at[pl.ds(ebase, ept)], w_v)

        def gather(c, b):
            return pltpu.async_copy(
                g_sh.at[sidx_v.at[pl.ds(c * _KM, _KM)]], bufs[b], gsem[b])

        def scat_add(c, b):
            return pltpu.async_copy(
                bufs[b], acc_sh.at[didx_v.at[pl.ds(c * _KM, _KM)]], asem[b],
                add=True)

        def scale(c, b):
            buf = bufs[b]

            def grp(j, carry):
                w16 = w_v[pl.ds(c * _KM + j * 16, 16)]
                for i in range(16):
                    wsp = jnp.full((16,), w16[i], _f32)
                    row = buf.at[j * 16 + i]
                    for col in range(_H // 16):
                        v = row[pl.ds(col * 16, 16)]
                        row[pl.ds(col * 16, 16)] = v * wsp
                return carry
            lax.fori_loop(0, _KM // 16, grp, 0)

        def pair(k, carry):
            c0 = 2 * k
            h0 = gather(c0, 0)
            h1 = gather(c0 + 1, 1)
            h0.wait()
            scale(c0, 0)
            a0 = scat_add(c0, 0)
            h1.wait()
            scale(c0 + 1, 1)
            a1 = scat_add(c0 + 1, 1)
            a0.wait()
            a1.wait()
            return carry
        lax.fori_loop(0, nchunks // 2, pair, 0)
        plsc.subcore_barrier()

        off = 0
        while off < rps:
            csz = min(_KM, rps - off)
            pltpu.sync_copy(acc_sh.at[pl.ds(sid * rps + off, csz)],
                            out_hbm.at[cid, pl.ds(sid * rps + off, csz)])
            off += csz

    return msg_kernel


def _make_cover_kernel():
    nout = _NW * _CLPT
    mesh = plsc.VectorSubcoreMesh(core_axis_name="c", subcore_axis_name="s")

    @functools.partial(
        pl.kernel,
        out_type=(jax.ShapeDtypeStruct((nout, _H), _f32),
                  jax.ShapeDtypeStruct((nout, _H), _f32)),
        mesh=mesh,
        compiler_params=_SC_PARAMS,
        scratch_types=[
            pltpu.VMEM((_CK,), _i32),
            pltpu.VMEM((_CK,), _i32),
            pltpu.VMEM((_CK, _H), _f32),
            pltpu.VMEM((_CLPT, _H), _f32),
            pltpu.VMEM((_CLPT, _H), _f32),
            pltpu.SemaphoreType.DMA,
        ],
    )
    def cover_kernel(h_hbm, cc_hbm, cn_hbm, out_add, out_max,
                     cc_v, cn_v, rows_v, asum, amax, sem):
        cid = lax.axis_index("c")
        sid = lax.axis_index("s")
        wid = sid * _NCORE + cid
        c_lo = wid * _CLPT
        c_hi = jnp.minimum(c_lo + _CLPT, _N1)

        def cnt_chunk(k, carry):
            lo, hi = carry
            pltpu.sync_copy(cc_hbm.at[pl.ds(k * _CK, _CK)], cc_v)

            def cnt_grp(j, carry2):
                lo2, hi2 = carry2
                v = cc_v[pl.ds(j * 16, 16)]
                lo2 = lo2 + plsc.all_reduce_population_count(v < c_lo)
                hi2 = hi2 + plsc.all_reduce_population_count(v < c_hi)
                return lo2, hi2
            return lax.fori_loop(0, _CK // 16, cnt_grp, (lo, hi))
        z16 = jnp.zeros((16,), _i32)
        lo_v, hi_v = lax.fori_loop(0, _CP // _CK, cnt_chunk, (z16, z16))
        e_lo = jnp.max(lo_v)
        e_hi = jnp.max(hi_v)
        estart = (e_lo // 8) * 8
        nch = (e_hi - estart + _CK - 1) // _CK

        _zero2d(asum, _CLPT, _H)
        _zero2d(amax, _CLPT, _H)
        it = _iota16()

        def chunk(k, carry):
            base = estart + k * _CK
            pltpu.sync_copy(cc_hbm.at[pl.ds(base, _CK)], cc_v)
            pltpu.sync_copy(cn_hbm.at[pl.ds(base, _CK)], cn_v)
            pltpu.async_copy(h_hbm.at[cn_v], rows_v, sem).wait()

            def ent(j, carry2):
                ci16 = cc_v[pl.ds(j * 16, 16)]
                for i in range(16):
                    ci = ci16[i]
                    inr = jnp.logical_and(ci >= c_lo, ci < c_hi)
                    l = jnp.clip(ci - c_lo, 0, _CLPT - 1)
                    m = jnp.where(inr, jnp.float32(1.0), jnp.float32(0.0))
                    rsel = jnp.full((16,), j * 16 + i, _i32)
                    lsel = jnp.full((16,), l, _i32)
                    for c in range(_H // 16):
                        cols = it + c * 16
                        val = plsc.load_gather(rows_v, [rsel, cols]) * m
                        so = plsc.load_gather(asum, [lsel, cols])
                        plsc.store_scatter(asum, [lsel, cols], so + val)
                        mo = plsc.load_gather(amax, [lsel, cols])
                        plsc.store_scatter(amax, [lsel, cols],
                                           jnp.maximum(mo, val))
                return carry2
            lax.fori_loop(0, _CK // 16, ent, 0)
            return carry
        lax.fori_loop(0, nch, chunk, 0)

        pltpu.sync_copy(asum, out_add.at[pl.ds(c_lo, _CLPT)])
        pltpu.sync_copy(amax, out_max.at[pl.ds(c_lo, _CLPT)])

    return cover_kernel


def _mm_first_body(x_ref, w_ref, degp_ref, g_ref, dinv_ref):
    dinv = lax.rsqrt(1.0 + degp_ref[0, :, 0:1] + degp_ref[1, :, 0:1])
    g_ref[...] = dinv * jnp.dot(x_ref[...], w_ref[...],
                                preferred_element_type=_f32)
    dinv_ref[...] = dinv


def _mm_first(x, w, degp, n):
    grid = pl.cdiv(n, _R)
    return pl.pallas_call(
        _mm_first_body,
        grid=(grid,),
        in_specs=[
            pl.BlockSpec((_R, x.shape[1]), lambda i: (i, 0)),
            pl.BlockSpec(w.shape, lambda i: (0, 0)),
            pl.BlockSpec((_NCORE, _R, 16), lambda i: (0, i, 0)),
        ],
        out_specs=[
            pl.BlockSpec((_R, _H), lambda i: (i, 0)),
            pl.BlockSpec((_R, 1), lambda i: (i, 0)),
        ],
        out_shape=[
            jax.ShapeDtypeStruct((n, _H), _f32),
            jax.ShapeDtypeStruct((n, 1), _f32),
        ],
    )(x, w, degp)


def _mm_pair_body(xa_ref, xb_ref, wt_ref, wb_ref, degp_ref, g_ref, dinv_ref):
    dinv = lax.rsqrt(1.0 + degp_ref[0, :, 0:1] + degp_ref[1, :, 0:1])
    acc = (jnp.dot(xa_ref[...], wt_ref[...], preferred_element_type=_f32)
           + jnp.dot(xb_ref[...], wb_ref[...], preferred_element_type=_f32))
    g_ref[...] = dinv * acc
    dinv_ref[...] = dinv


def _mm_pair(xa, xb, wt, wb, degp, n):
    grid = pl.cdiv(n, _R)
    return pl.pallas_call(
        _mm_pair_body,
        grid=(grid,),
        in_specs=[
            pl.BlockSpec((_R, _H), lambda i: (i, 0)),
            pl.BlockSpec((_R, _H), lambda i: (i, 0)),
            pl.BlockSpec((_H, _H), lambda i: (0, 0)),
            pl.BlockSpec((_H, _H), lambda i: (0, 0)),
            pl.BlockSpec((_NCORE, _R, 16), lambda i: (0, i, 0)),
        ],
        out_specs=[
            pl.BlockSpec((_R, _H), lambda i: (i, 0)),
            pl.BlockSpec((_R, 1), lambda i: (i, 0)),
        ],
        out_shape=[
            jax.ShapeDtypeStruct((n, _H), _f32),
            jax.ShapeDtypeStruct((n, 1), _f32),
        ],
    )(xa, xb, wt, wb, degp)


def _mid_body(msgp_ref, g_ref, dinv_ref, b_ref, w_ref, h_ref, g2_ref):
    dinv = dinv_ref[...]
    h = jnp.maximum(
        dinv * (msgp_ref[0] + msgp_ref[1] + g_ref[...]) + b_ref[...], 0.0)
    h_ref[...] = h
    g2_ref[...] = dinv * jnp.dot(h, w_ref[...], preferred_element_type=_f32)


def _mid(msgp, g, dinv, b, w, n):
    grid = pl.cdiv(n, _R)
    return pl.pallas_call(
        _mid_body,
        grid=(grid,),
        in_specs=[
            pl.BlockSpec((_NCORE, _R, _H), lambda i: (0, i, 0)),
            pl.BlockSpec((_R, _H), lambda i: (i, 0)),
            pl.BlockSpec((_R, 1), lambda i: (i, 0)),
            pl.BlockSpec((1, _H), lambda i: (0, 0)),
            pl.BlockSpec((_H, _H), lambda i: (0, 0)),
        ],
        out_specs=[
            pl.BlockSpec((_R, _H), lambda i: (i, 0)),
            pl.BlockSpec((_R, _H), lambda i: (i, 0)),
        ],
        out_shape=[
            jax.ShapeDtypeStruct((n, _H), _f32),
            jax.ShapeDtypeStruct((n, _H), _f32),
        ],
    )(msgp, g, dinv, b, w)


def _blockout_body(msgp_ref, g2_ref, dinv_ref, b2_ref, h1_ref,
                   wlt_ref, wlb_ref, bl_ref, h_ref):
    dinv = dinv_ref[...]
    h2 = jnp.maximum(
        dinv * (msgp_ref[0] + msgp_ref[1] + g2_ref[...]) + b2_ref[...], 0.0)
    acc = (jnp.dot(h1_ref[...], wlt_ref[...], preferred_element_type=_f32)
           + jnp.dot(h2, wlb_ref[...], preferred_element_type=_f32))
    h_ref[...] = jnp.maximum(acc + bl_ref[...], 0.0)


def _blockout(msgp, g2, dinv, b2, h1, wlt, wlb, bl, n):
    grid = pl.cdiv(n, _R)
    return pl.pallas_call(
        _blockout_body,
        grid=(grid,),
        in_specs=[
            pl.BlockSpec((_NCORE, _R, _H), lambda i: (0, i, 0)),
            pl.BlockSpec((_R, _H), lambda i: (i, 0)),
            pl.BlockSpec((_R, 1), lambda i: (i, 0)),
            pl.BlockSpec((1, _H), lambda i: (0, 0)),
            pl.BlockSpec((_R, _H), lambda i: (i, 0)),
            pl.BlockSpec((_H, _H), lambda i: (0, 0)),
            pl.BlockSpec((_H, _H), lambda i: (0, 0)),
            pl.BlockSpec((1, _H), lambda i: (0, 0)),
        ],
        out_specs=pl.BlockSpec((_R, _H), lambda i: (i, 0)),
        out_shape=jax.ShapeDtypeStruct((n, _H), _f32),
    )(msgp, g2, dinv, b2, h1, wlt, wlb, bl)


def _pool_body(n, h_ref, bidx_ref, sum_ref, max_ref):
    i = pl.program_id(0)
    h = h_ref[...]
    bid = bidx_ref[...]
    row = i * _R + lax.broadcasted_iota(_i32, (_R, 1), 0)
    valid = row < n
    onehot = jnp.logical_and(
        bid == lax.broadcasted_iota(_i32, (1, _B), 1), valid).astype(_f32)
    s = lax.dot_general(onehot, h, (((0,), (0,)), ((), ())),
                        preferred_element_type=_f32)
    rows = []
    for b in range(_B):
        m = jnp.where(jnp.logical_and(bid == b, valid), h, 0.0)
        rows.append(jnp.max(m, axis=0, keepdims=True))
    mx = jnp.concatenate(rows, axis=0)

    @pl.when(i == 0)
    def _():
        sum_ref[...] = s
        max_ref[...] = mx

    @pl.when(i > 0)
    def _():
        sum_ref[...] = sum_ref[...] + s
        max_ref[...] = jnp.maximum(max_ref[...], mx)


def _pool(h, bidx, n):
    grid = pl.cdiv(n, _R)
    return pl.pallas_call(
        functools.partial(_pool_body, n),
        grid=(grid,),
        in_specs=[
            pl.BlockSpec((_R, _H), lambda i: (i, 0)),
            pl.BlockSpec((_R, 1), lambda i: (i, 0)),
        ],
        out_specs=[
            pl.BlockSpec((_B, _H), lambda i: (0, 0)),
            pl.BlockSpec((_B, _H), lambda i: (0, 0)),
        ],
        out_shape=[
            jax.ShapeDtypeStruct((_B, _H), _f32),
            jax.ShapeDtypeStruct((_B, _H), _f32),
        ],
    )(h, bidx)


def _head_body(x0, x1, x2, x3, gamma, beta, w1, b1, w2, b2, out):
    z = jnp.concatenate([x0[...], x1[...], x2[...], x3[...]], axis=1)
    mu = jnp.mean(z, axis=0, keepdims=True)
    var = jnp.mean((z - mu) ** 2, axis=0, keepdims=True)
    z = (z - mu) * lax.rsqrt(var + 1e-5) * gamma[...] + beta[...]
    z = jnp.maximum(jnp.dot(z, w1[...], preferred_element_type=_f32)
                    + b1[...], 0.0)
    z = jnp.dot(z, w2[...], preferred_element_type=_f32) + b2[...]
    z = z - jnp.max(z, axis=1, keepdims=True)
    ez = jnp.exp(z)
    out[...] = ez / jnp.sum(ez, axis=1, keepdims=True)


def _head(x0, x1, x2, x3, gamma, beta, w1, b1, w2, b2):
    return pl.pallas_call(
        _head_body,
        out_shape=jax.ShapeDtypeStruct((_B, _NC), _f32),
    )(x0, x1, x2, x3, gamma, beta, w1, b1, w2, b2)


_deg0 = _make_deg_kernel(_N0, _EP0)
_deg1 = _make_deg_kernel(_N1, _EP1)
_msg0 = _make_msg_kernel(_N0, _EP0)
_msg1 = _make_msg_kernel(_N1, _EP1)
_cover = _make_cover_kernel()


def kernel(x, edge_index, edge_weight, batch, cover_node, cover_cluster,
           edge_index2, edge_weight2, batch2, cW1, cb1, cW2, cb2, cWl, cbl,
           bW1, bb1, bW2, bb2, bWl, bbl, gamma, beta, l1W, l1b, l2W, l2b):
    s0 = jnp.pad(edge_index[0], (0, _EP0 - _E0))
    d0 = jnp.pad(edge_index[1], (0, _EP0 - _E0))
    w0 = jnp.pad(edge_weight, (0, _EP0 - _E0))
    s1 = jnp.pad(edge_index2[0], (0, _EP1 - _E1))
    d1 = jnp.pad(edge_index2[1], (0, _EP1 - _E1))
    w1 = jnp.pad(edge_weight2, (0, _EP1 - _E1))
    ccp = jnp.pad(cover_cluster, (0, _CP - _C), constant_values=1 << 30)
    cnp = jnp.pad(cover_node, (0, _CP - _C))

    degp0 = _deg0(d0, w0)
    degp1 = _deg1(d1, w1)

    g1, dinv0 = _mm_first(x, cW1, degp0, _N0)
    mp = _msg0(jnp.pad(g1, ((0, _NP0 - _N0), (0, 0))), s0, d0, w0)
    h1, g2 = _mid(mp, g1, dinv0, cb1.reshape(1, _H), cW2, _N0)
    mp = _msg0(jnp.pad(g2, ((0, _NP0 - _N0), (0, 0))), s0, d0, w0)
    h = _blockout(mp, g2, dinv0, cb2.reshape(1, _H), h1,
                  cWl[:_H], cWl[_H:], cbl.reshape(1, _H), _N0)

    xs0, xs1 = _pool(h, batch.reshape(_N0, 1), _N0)
    xadd_p, xmax_p = _cover(h, ccp, cnp)
    x_add = xadd_p[:_N1]
    x_max = xmax_p[:_N1]

    gB, dinv1 = _mm_pair(x_add, x_max, bW1[:_H], bW1[_H:], degp1, _N1)
    mp = _msg1(jnp.pad(gB, ((0, _NP1 - _N1), (0, 0))), s1, d1, w1)
    h1B, g2B = _mid(mp, gB, dinv1, bb1.reshape(1, _H), bW2, _N1)
    mp = _msg1(jnp.pad(g2B, ((0, _NP1 - _N1), (0, 0))), s1, d1, w1)
    hB = _blockout(mp, g2B, dinv1, bb2.reshape(1, _H), h1B,
                   bWl[:_H], bWl[_H:], bbl.reshape(1, _H), _N1)

    xs2, xs3 = _pool(hB, batch2.reshape(_N1, 1), _N1)

    return _head(xs0, xs1, xs2, xs3,
                 gamma.reshape(1, 4 * _H), beta.reshape(1, 4 * _H),
                 l1W, l1b.reshape(1, _H), l2W, l2b.reshape(1, _NC))


# speedup vs baseline: 20.6892x; 1.0456x over previous
import functools

import jax
import jax.numpy as jnp
from jax import lax
from jax.experimental import pallas as pl
from jax.experimental.pallas import tpu as pltpu
from jax.experimental.pallas import tpu_sc as plsc

_N0, _E0, _D, _H, _B = 10000, 320000, 128, 64, 16
_C, _N1, _E1, _NC = 15000, 2500, 80000, 10
_NCORE, _NSUB, _NW = 2, 16, 32
_K = 512
_KM = 128
_CK = 256
_CLPT = 80
_R = 512

_EP0 = 327680
_EP1 = 81920
_CP = 15360
_NP0 = 10112
_NP1 = 2560

_f32 = jnp.float32
_i32 = jnp.int32


def _iota16():
    return lax.iota(_i32, 16)


def _zero2d(ref, nrows, ncols):
    z = jnp.zeros((16,), _f32)
    it = _iota16()

    def zrow(j, carry):
        rows16 = it + j * 16
        for c in range(ncols):
            plsc.store_scatter(ref, [rows16, jnp.full((16,), c, _i32)], z)
        return carry
    lax.fori_loop(0, nrows // 16, zrow, 0)


_SC_PARAMS = pltpu.CompilerParams(needs_layout_passes=False,
                                  use_tc_tiling_on_sc=False)


def _round_up(a, m):
    return ((a + m - 1) // m) * m


def _make_deg_kernel(n, e_pad):
    npad = _round_up(n, _NSUB * 8)
    rps = npad // _NSUB
    ept = e_pad // _NW
    nchunks = ept // _K
    mesh = plsc.VectorSubcoreMesh(core_axis_name="c", subcore_axis_name="s")

    @functools.partial(
        pl.kernel,
        out_type=jax.ShapeDtypeStruct((_NCORE, npad, 16), _f32),
        mesh=mesh,
        compiler_params=_SC_PARAMS,
        scratch_types=[
            pltpu.VMEM((ept,), _i32),
            pltpu.VMEM((ept,), _f32),
            pltpu.VMEM((_K, 16), _f32),
            pltpu.VMEM_SHARED((npad, 16), _f32),
        ],
    )
    def deg_kernel(d_hbm, w_hbm, out_hbm, didx_v, w_v, rows_v, acc_sh):
        cid = lax.axis_index("c")
        sid = lax.axis_index("s")
        wid = sid * _NCORE + cid

        _zero2d(rows_v, _K, 16)
        off = 0
        while off < rps:
            csz = min(_K, rps - off)
            pltpu.sync_copy(rows_v.at[pl.ds(0, csz)],
                            acc_sh.at[pl.ds(sid * rps + off, csz)])
            off += csz
        plsc.subcore_barrier()

        ebase = wid * ept
        pltpu.sync_copy(d_hbm.at[pl.ds(ebase, ept)], didx_v)
        pltpu.sync_copy(w_hbm.at[pl.ds(ebase, ept)], w_v)

        def chunk(k, carry):
            def grp(j, carry2):
                w16 = w_v[pl.ds(k * _K + j * 16, 16)]
                for i in range(16):
                    row = rows_v.at[j * 16 + i]
                    row[pl.ds(0, 16)] = jnp.full((16,), w16[i], _f32)
                return carry2
            lax.fori_loop(0, _K // 16, grp, 0)
            pltpu.sync_copy(rows_v,
                            acc_sh.at[didx_v.at[pl.ds(k * _K, _K)]], add=True)
            return carry
        lax.fori_loop(0, nchunks, chunk, 0)
        plsc.subcore_barrier()

        off = 0
        while off < rps:
            csz = min(_K, rps - off)
            pltpu.sync_copy(acc_sh.at[pl.ds(sid * rps + off, csz)],
                            out_hbm.at[cid, pl.ds(sid * rps + off, csz)])
            off += csz

    return deg_kernel


def _make_msg_kernel(n, e_pad):
    npad = _round_up(n, _NSUB * 8)
    rps = npad // _NSUB
    ept = e_pad // _NW
    nchunks = ept // _KM
    mesh = plsc.VectorSubcoreMesh(core_axis_name="c", subcore_axis_name="s")

    @functools.partial(
        pl.kernel,
        out_type=jax.ShapeDtypeStruct((_NCORE, npad, _H), _f32),
        mesh=mesh,
        compiler_params=_SC_PARAMS,
        scratch_types=[
            pltpu.VMEM((ept,), _i32),
            pltpu.VMEM((ept,), _i32),
            pltpu.VMEM((ept,), _f32),
            pltpu.VMEM((_KM, _H), _f32),
            pltpu.VMEM((_KM, _H), _f32),
            pltpu.VMEM_SHARED((npad, _H), _f32),
            pltpu.VMEM_SHARED((npad, _H), _f32),
            pltpu.SemaphoreType.DMA,
            pltpu.SemaphoreType.DMA,
            pltpu.SemaphoreType.DMA,
            pltpu.SemaphoreType.DMA,
        ],
    )
    def msg_kernel(g_hbm, s_hbm, d_hbm, w_hbm, out_hbm,
                   sidx_v, didx_v, w_v, buf0, buf1, acc_sh, g_sh,
                   gsem0, gsem1, asem0, asem1):
        cid = lax.axis_index("c")
        sid = lax.axis_index("s")
        wid = sid * _NCORE + cid
        bufs = (buf0, buf1)
        gsem = (gsem0, gsem1)
        asem = (asem0, asem1)

        pltpu.sync_copy(g_hbm.at[pl.ds(sid * rps, rps)],
                        g_sh.at[pl.ds(sid * rps, rps)])

        _zero2d(buf0, _KM, _H)
        off = 0
        while off < rps:
            csz = min(_KM, rps - off)
            pltpu.sync_copy(buf0.at[pl.ds(0, csz)],
                            acc_sh.at[pl.ds(sid * rps + off, csz)])
            off += csz
        plsc.subcore_barrier()

        ebase = wid * ept
        pltpu.sync_copy(s_hbm.at[pl.ds(ebase, ept)], sidx_v)
        pltpu.sync_copy(d_hbm.at[pl.ds(ebase, ept)], didx_v)
        pltpu.sync_copy(w_hbm.at[pl.ds(ebase, ept)], w_v)

        def gather(c, b):
            return pltpu.async_copy(
                g_sh.at[sidx_v.at[pl.ds(c * _KM, _KM)]], bufs[b], gsem[b])

        def scat_add(c, b):
            return pltpu.async_copy(
                bufs[b], acc_sh.at[didx_v.at[pl.ds(c * _KM, _KM)]], asem[b],
                add=True)

        def scale(c, b):
            buf = bufs[b]

            def grp(j, carry):
                w16 = w_v[pl.ds(c * _KM + j * 16, 16)]
                for i in range(16):
                    wsp = jnp.full((16,), w16[i], _f32)
                    row = buf.at[j * 16 + i]
                    for col in range(_H // 16):
                        v = row[pl.ds(col * 16, 16)]
                        row[pl.ds(col * 16, 16)] = v * wsp
                return carry
            lax.fori_loop(0, _KM // 16, grp, 0)

        def pair(k, carry):
            c0 = 2 * k
            h0 = gather(c0, 0)
            h1 = gather(c0 + 1, 1)
            h0.wait()
            scale(c0, 0)
            a0 = scat_add(c0, 0)
            h1.wait()
            scale(c0 + 1, 1)
            a1 = scat_add(c0 + 1, 1)
            a0.wait()
            a1.wait()
            return carry
        lax.fori_loop(0, nchunks // 2, pair, 0)
        plsc.subcore_barrier()

        off = 0
        while off < rps:
            csz = min(_KM, rps - off)
            pltpu.sync_copy(acc_sh.at[pl.ds(sid * rps + off, csz)],
                            out_hbm.at[cid, pl.ds(sid * rps + off, csz)])
            off += csz

    return msg_kernel


def _make_cover_kernel():
    nout = _NW * _CLPT
    mesh = plsc.VectorSubcoreMesh(core_axis_name="c", subcore_axis_name="s")

    @functools.partial(
        pl.kernel,
        out_type=(jax.ShapeDtypeStruct((nout, _H), _f32),
                  jax.ShapeDtypeStruct((nout, _H), _f32)),
        mesh=mesh,
        compiler_params=_SC_PARAMS,
        scratch_types=[
            pltpu.VMEM((_CP,), _i32),
            pltpu.VMEM((_CP,), _i32),
            pltpu.VMEM((_CK, _H), _f32),
            pltpu.VMEM((_CLPT, _H), _f32),
            pltpu.VMEM((_CLPT, _H), _f32),
            pltpu.SemaphoreType.DMA,
        ],
    )
    def cover_kernel(h_hbm, cc_hbm, cn_hbm, out_add, out_max,
                     cc_v, cn_v, rows_v, asum, amax, sem):
        cid = lax.axis_index("c")
        sid = lax.axis_index("s")
        wid = sid * _NCORE + cid
        c_lo = wid * _CLPT
        c_hi = jnp.minimum(c_lo + _CLPT, _N1)

        pltpu.sync_copy(cc_hbm, cc_v)
        pltpu.sync_copy(cn_hbm, cn_v)

        def cnt_grp(j, carry2):
            lo2, hi2 = carry2
            v = cc_v[pl.ds(j * 16, 16)]
            lo2 = lo2 + plsc.all_reduce_population_count(v < c_lo)
            hi2 = hi2 + plsc.all_reduce_population_count(v < c_hi)
            return lo2, hi2
        z16 = jnp.zeros((16,), _i32)
        lo_v, hi_v = lax.fori_loop(0, _CP // 16, cnt_grp, (z16, z16))
        e_lo = jnp.max(lo_v)
        e_hi = jnp.max(hi_v)
        estart = (e_lo // 16) * 16
        nch = (e_hi - estart + _CK - 1) // _CK

        _zero2d(asum, _CLPT, _H)
        _zero2d(amax, _CLPT, _H)
        it = _iota16()

        def chunk(k, carry):
            base = estart + k * _CK
            pltpu.async_copy(h_hbm.at[cn_v.at[pl.ds(base, _CK)]],
                             rows_v, sem).wait()

            def ent(j, carry2):
                ci16 = cc_v[pl.ds(base + j * 16, 16)]
                for i in range(16):
                    ci = ci16[i]
                    inr = jnp.logical_and(ci >= c_lo, ci < c_hi)
                    l = jnp.clip(ci - c_lo, 0, _CLPT - 1)
                    m = jnp.where(inr, jnp.float32(1.0), jnp.float32(0.0))
                    rsel = jnp.full((16,), j * 16 + i, _i32)
                    lsel = jnp.full((16,), l, _i32)
                    for c in range(_H // 16):
                        cols = it + c * 16
                        val = plsc.load_gather(rows_v, [rsel, cols]) * m
                        so = plsc.load_gather(asum, [lsel, cols])
                        plsc.store_scatter(asum, [lsel, cols], so + val)
                        mo = plsc.load_gather(amax, [lsel, cols])
                        plsc.store_scatter(amax, [lsel, cols],
                                           jnp.maximum(mo, val))
                return carry2
            lax.fori_loop(0, _CK // 16, ent, 0)
            return carry
        lax.fori_loop(0, nch, chunk, 0)

        pltpu.sync_copy(asum, out_add.at[pl.ds(c_lo, _CLPT)])
        pltpu.sync_copy(amax, out_max.at[pl.ds(c_lo, _CLPT)])

    return cover_kernel


def _mm_first_body(x_ref, w_ref, degp_ref, g_ref, dinv_ref):
    dinv = lax.rsqrt(1.0 + degp_ref[0, :, 0:1] + degp_ref[1, :, 0:1])
    g_ref[...] = dinv * jnp.dot(x_ref[...], w_ref[...],
                                preferred_element_type=_f32)
    dinv_ref[...] = dinv


def _mm_first(x, w, degp, n):
    grid = pl.cdiv(n, _R)
    return pl.pallas_call(
        _mm_first_body,
        grid=(grid,),
        in_specs=[
            pl.BlockSpec((_R, x.shape[1]), lambda i: (i, 0)),
            pl.BlockSpec(w.shape, lambda i: (0, 0)),
            pl.BlockSpec((_NCORE, _R, 16), lambda i: (0, i, 0)),
        ],
        out_specs=[
            pl.BlockSpec((_R, _H), lambda i: (i, 0)),
            pl.BlockSpec((_R, 1), lambda i: (i, 0)),
        ],
        out_shape=[
            jax.ShapeDtypeStruct((n, _H), _f32),
            jax.ShapeDtypeStruct((n, 1), _f32),
        ],
    )(x, w, degp)


def _mm_pair_body(xa_ref, xb_ref, wt_ref, wb_ref, degp_ref, g_ref, dinv_ref):
    dinv = lax.rsqrt(1.0 + degp_ref[0, :, 0:1] + degp_ref[1, :, 0:1])
    acc = (jnp.dot(xa_ref[...], wt_ref[...], preferred_element_type=_f32)
           + jnp.dot(xb_ref[...], wb_ref[...], preferred_element_type=_f32))
    g_ref[...] = dinv * acc
    dinv_ref[...] = dinv


def _mm_pair(xa, xb, wt, wb, degp, n):
    grid = pl.cdiv(n, _R)
    return pl.pallas_call(
        _mm_pair_body,
        grid=(grid,),
        in_specs=[
            pl.BlockSpec((_R, _H), lambda i: (i, 0)),
            pl.BlockSpec((_R, _H), lambda i: (i, 0)),
            pl.BlockSpec((_H, _H), lambda i: (0, 0)),
            pl.BlockSpec((_H, _H), lambda i: (0, 0)),
            pl.BlockSpec((_NCORE, _R, 16), lambda i: (0, i, 0)),
        ],
        out_specs=[
            pl.BlockSpec((_R, _H), lambda i: (i, 0)),
            pl.BlockSpec((_R, 1), lambda i: (i, 0)),
        ],
        out_shape=[
            jax.ShapeDtypeStruct((n, _H), _f32),
            jax.ShapeDtypeStruct((n, 1), _f32),
        ],
    )(xa, xb, wt, wb, degp)


def _mid_body(msgp_ref, g_ref, dinv_ref, b_ref, w_ref, h_ref, g2_ref):
    dinv = dinv_ref[...]
    h = jnp.maximum(
        dinv * (msgp_ref[0] + msgp_ref[1] + g_ref[...]) + b_ref[...], 0.0)
    h_ref[...] = h
    g2_ref[...] = dinv * jnp.dot(h, w_ref[...], preferred_element_type=_f32)


def _mid(msgp, g, dinv, b, w, n):
    grid = pl.cdiv(n, _R)
    return pl.pallas_call(
        _mid_body,
        grid=(grid,),
        in_specs=[
            pl.BlockSpec((_NCORE, _R, _H), lambda i: (0, i, 0)),
            pl.BlockSpec((_R, _H), lambda i: (i, 0)),
            pl.BlockSpec((_R, 1), lambda i: (i, 0)),
            pl.BlockSpec((1, _H), lambda i: (0, 0)),
            pl.BlockSpec((_H, _H), lambda i: (0, 0)),
        ],
        out_specs=[
            pl.BlockSpec((_R, _H), lambda i: (i, 0)),
            pl.BlockSpec((_R, _H), lambda i: (i, 0)),
        ],
        out_shape=[
            jax.ShapeDtypeStruct((n, _H), _f32),
            jax.ShapeDtypeStruct((n, _H), _f32),
        ],
    )(msgp, g, dinv, b, w)


def _blockout_body(msgp_ref, g2_ref, dinv_ref, b2_ref, h1_ref,
                   wlt_ref, wlb_ref, bl_ref, h_ref):
    dinv = dinv_ref[...]
    h2 = jnp.maximum(
        dinv * (msgp_ref[0] + msgp_ref[1] + g2_ref[...]) + b2_ref[...], 0.0)
    acc = (jnp.dot(h1_ref[...], wlt_ref[...], preferred_element_type=_f32)
           + jnp.dot(h2, wlb_ref[...], preferred_element_type=_f32))
    h_ref[...] = jnp.maximum(acc + bl_ref[...], 0.0)


def _blockout(msgp, g2, dinv, b2, h1, wlt, wlb, bl, n):
    grid = pl.cdiv(n, _R)
    return pl.pallas_call(
        _blockout_body,
        grid=(grid,),
        in_specs=[
            pl.BlockSpec((_NCORE, _R, _H), lambda i: (0, i, 0)),
            pl.BlockSpec((_R, _H), lambda i: (i, 0)),
            pl.BlockSpec((_R, 1), lambda i: (i, 0)),
            pl.BlockSpec((1, _H), lambda i: (0, 0)),
            pl.BlockSpec((_R, _H), lambda i: (i, 0)),
            pl.BlockSpec((_H, _H), lambda i: (0, 0)),
            pl.BlockSpec((_H, _H), lambda i: (0, 0)),
            pl.BlockSpec((1, _H), lambda i: (0, 0)),
        ],
        out_specs=pl.BlockSpec((_R, _H), lambda i: (i, 0)),
        out_shape=jax.ShapeDtypeStruct((n, _H), _f32),
    )(msgp, g2, dinv, b2, h1, wlt, wlb, bl)


def _pool_body(n, h_ref, bidx_ref, sum_ref, max_ref):
    i = pl.program_id(0)
    h = h_ref[...]
    bid = bidx_ref[...]
    row = i * _R + lax.broadcasted_iota(_i32, (_R, 1), 0)
    valid = row < n
    onehot = jnp.logical_and(
        bid == lax.broadcasted_iota(_i32, (1, _B), 1), valid).astype(_f32)
    s = lax.dot_general(onehot, h, (((0,), (0,)), ((), ())),
                        preferred_element_type=_f32)
    rows = []
    for b in range(_B):
        m = jnp.where(jnp.logical_and(bid == b, valid), h, 0.0)
        rows.append(jnp.max(m, axis=0, keepdims=True))
    mx = jnp.concatenate(rows, axis=0)

    @pl.when(i == 0)
    def _():
        sum_ref[...] = s
        max_ref[...] = mx

    @pl.when(i > 0)
    def _():
        sum_ref[...] = sum_ref[...] + s
        max_ref[...] = jnp.maximum(max_ref[...], mx)


def _pool(h, bidx, n):
    grid = pl.cdiv(n, _R)
    return pl.pallas_call(
        functools.partial(_pool_body, n),
        grid=(grid,),
        in_specs=[
            pl.BlockSpec((_R, _H), lambda i: (i, 0)),
            pl.BlockSpec((_R, 1), lambda i: (i, 0)),
        ],
        out_specs=[
            pl.BlockSpec((_B, _H), lambda i: (0, 0)),
            pl.BlockSpec((_B, _H), lambda i: (0, 0)),
        ],
        out_shape=[
            jax.ShapeDtypeStruct((_B, _H), _f32),
            jax.ShapeDtypeStruct((_B, _H), _f32),
        ],
    )(h, bidx)


def _head_body(x0, x1, x2, x3, gamma, beta, w1, b1, w2, b2, out):
    z = jnp.concatenate([x0[...], x1[...], x2[...], x3[...]], axis=1)
    mu = jnp.mean(z, axis=0, keepdims=True)
    var = jnp.mean((z - mu) ** 2, axis=0, keepdims=True)
    z = (z - mu) * lax.rsqrt(var + 1e-5) * gamma[...] + beta[...]
    z = jnp.maximum(jnp.dot(z, w1[...], preferred_element_type=_f32)
                    + b1[...], 0.0)
    z = jnp.dot(z, w2[...], preferred_element_type=_f32) + b2[...]
    z = z - jnp.max(z, axis=1, keepdims=True)
    ez = jnp.exp(z)
    out[...] = ez / jnp.sum(ez, axis=1, keepdims=True)


def _head(x0, x1, x2, x3, gamma, beta, w1, b1, w2, b2):
    return pl.pallas_call(
        _head_body,
        out_shape=jax.ShapeDtypeStruct((_B, _NC), _f32),
    )(x0, x1, x2, x3, gamma, beta, w1, b1, w2, b2)


_deg0 = _make_deg_kernel(_N0, _EP0)
_deg1 = _make_deg_kernel(_N1, _EP1)
_msg0 = _make_msg_kernel(_N0, _EP0)
_msg1 = _make_msg_kernel(_N1, _EP1)
_cover = _make_cover_kernel()


def kernel(x, edge_index, edge_weight, batch, cover_node, cover_cluster,
           edge_index2, edge_weight2, batch2, cW1, cb1, cW2, cb2, cWl, cbl,
           bW1, bb1, bW2, bb2, bWl, bbl, gamma, beta, l1W, l1b, l2W, l2b):
    s0 = jnp.pad(edge_index[0], (0, _EP0 - _E0))
    d0 = jnp.pad(edge_index[1], (0, _EP0 - _E0))
    w0 = jnp.pad(edge_weight, (0, _EP0 - _E0))
    s1 = jnp.pad(edge_index2[0], (0, _EP1 - _E1))
    d1 = jnp.pad(edge_index2[1], (0, _EP1 - _E1))
    w1 = jnp.pad(edge_weight2, (0, _EP1 - _E1))
    ccp = jnp.pad(cover_cluster, (0, _CP - _C), constant_values=1 << 30)
    cnp = jnp.pad(cover_node, (0, _CP - _C))

    degp0 = _deg0(d0, w0)
    degp1 = _deg1(d1, w1)

    g1, dinv0 = _mm_first(x, cW1, degp0, _N0)
    mp = _msg0(jnp.pad(g1, ((0, _NP0 - _N0), (0, 0))), s0, d0, w0)
    h1, g2 = _mid(mp, g1, dinv0, cb1.reshape(1, _H), cW2, _N0)
    mp = _msg0(jnp.pad(g2, ((0, _NP0 - _N0), (0, 0))), s0, d0, w0)
    h = _blockout(mp, g2, dinv0, cb2.reshape(1, _H), h1,
                  cWl[:_H], cWl[_H:], cbl.reshape(1, _H), _N0)

    xs0, xs1 = _pool(h, batch.reshape(_N0, 1), _N0)
    xadd_p, xmax_p = _cover(h, ccp, cnp)
    x_add = xadd_p[:_N1]
    x_max = xmax_p[:_N1]

    gB, dinv1 = _mm_pair(x_add, x_max, bW1[:_H], bW1[_H:], degp1, _N1)
    mp = _msg1(jnp.pad(gB, ((0, _NP1 - _N1), (0, 0))), s1, d1, w1)
    h1B, g2B = _mid(mp, gB, dinv1, bb1.reshape(1, _H), bW2, _N1)
    mp = _msg1(jnp.pad(g2B, ((0, _NP1 - _N1), (0, 0))), s1, d1, w1)
    hB = _blockout(mp, g2B, dinv1, bb2.reshape(1, _H), h1B,
                   bWl[:_H], bWl[_H:], bbl.reshape(1, _H), _N1)

    xs2, xs3 = _pool(hB, batch2.reshape(_N1, 1), _N1)

    return _head(xs0, xs1, xs2, xs3,
                 gamma.reshape(1, 4 * _H), beta.reshape(1, 4 * _H),
                 l1W, l1b.reshape(1, _H), l2W, l2b.reshape(1, _NC))
